# Initial kernel scaffold; baseline (speedup 1.0000x reference)
import functools

import jax
import jax.numpy as jnp
from jax import lax
from jax.experimental import pallas as pl
from jax.experimental.pallas import tpu as pltpu
from jax.experimental.pallas import tpu_sc as plsc

N = 100000
E = 3200000
IN_CH = 896
HID = 16
OUT_CH = 640
LSTM_H = 32

NW = 32
K = 8
ROW = 128
RP = 25088
RW = RP // NW
EPAD = RP * ROW - E
DISCARD = 96
NACC = N + DISCARD
RPT = NACC // 16

_mesh = plsc.VectorSubcoreMesh(core_axis_name="c", subcore_axis_name="s")


@functools.partial(
    pl.kernel,
    out_type=jax.ShapeDtypeStruct((2 * NACC, HID), jnp.float32),
    mesh=_mesh,
    scratch_types=[
        pltpu.VMEM((K, ROW), jnp.int32),
        pltpu.VMEM((K, ROW), jnp.int32),
        pltpu.VMEM((K * ROW, HID), jnp.float32),
        pltpu.VMEM_SHARED((NACC, HID), jnp.float32),
        pltpu.SemaphoreType.DMA,
    ],
)
def _sc_gather_scatter(srcr, dstr, feat, zero, out, src_v, dst_v, rows_v, acc, sem):
    cid = lax.axis_index("c")
    sid = lax.axis_index("s")
    wid = sid * 2 + cid
    pltpu.sync_copy(zero.at[pl.ds(sid * RPT, RPT)], acc.at[pl.ds(sid * RPT, RPT)])
    plsc.subcore_barrier()
    base = wid * RW

    def body(g, carry):
        r0 = base + g * K
        pltpu.sync_copy(srcr.at[pl.ds(r0, K)], src_v)
        pltpu.sync_copy(dstr.at[pl.ds(r0, K)], dst_v)
        descs = []
        for j in range(K):
            descs.append(
                pltpu.async_copy(
                    feat.at[src_v.at[j]], rows_v.at[pl.ds(j * ROW, ROW)], sem
                )
            )
        for d in descs:
            d.wait()
        for j in range(K):
            pltpu.sync_copy(
                rows_v.at[pl.ds(j * ROW, ROW)], acc.at[dst_v.at[j]], add=True
            )
        return carry

    lax.fori_loop(0, RW // K, body, 0)
    plsc.subcore_barrier()
    pltpu.sync_copy(
        acc.at[pl.ds(sid * RPT, RPT)],
        out.at[pl.ds(cid * NACC + sid * RPT, RPT)],
    )


@functools.partial(
    pl.kernel,
    out_type=jax.ShapeDtypeStruct((2 * NACC, HID), jnp.float32),
    mesh=_mesh,
    scratch_types=[
        pltpu.VMEM((K, ROW), jnp.int32),
        pltpu.VMEM((ROW, HID), jnp.float32),
        pltpu.VMEM_SHARED((NACC, HID), jnp.float32),
    ],
)
def _sc_degree(dstr, ones, zero, out, dst_v, ones_v, acc):
    cid = lax.axis_index("c")
    sid = lax.axis_index("s")
    wid = sid * 2 + cid
    pltpu.sync_copy(zero.at[pl.ds(sid * RPT, RPT)], acc.at[pl.ds(sid * RPT, RPT)])
    pltpu.sync_copy(ones, ones_v)
    plsc.subcore_barrier()
    base = wid * RW

    def body(g, carry):
        r0 = base + g * K
        pltpu.sync_copy(dstr.at[pl.ds(r0, K)], dst_v)
        for j in range(K):
            pltpu.sync_copy(ones_v, acc.at[dst_v.at[j]], add=True)
        return carry

    lax.fori_loop(0, RW // K, body, 0)
    plsc.subcore_barrier()
    pltpu.sync_copy(
        acc.at[pl.ds(sid * RPT, RPT)],
        out.at[pl.ds(cid * NACC + sid * RPT, RPT)],
    )


BM = 2000
GRID = N // BM


def _tc0_body(x_ref, w0_ref, dega_ref, degb_ref, h0_ref, hp0_ref, dinv_ref):
    h0 = jnp.dot(x_ref[...], w0_ref[...], preferred_element_type=jnp.float32)
    deg = 1.0 + dega_ref[...] + degb_ref[...]
    dinv = lax.rsqrt(deg)
    h0_ref[...] = h0
    dinv_ref[...] = dinv
    hp0_ref[...] = h0 * dinv


def _tc0(x, w0, dega, degb):
    return pl.pallas_call(
        _tc0_body,
        grid=(GRID,),
        in_specs=[
            pl.BlockSpec((BM, IN_CH), lambda i: (i, 0)),
            pl.BlockSpec((IN_CH, HID), lambda i: (0, 0)),
            pl.BlockSpec((BM, HID), lambda i: (i, 0)),
            pl.BlockSpec((BM, HID), lambda i: (i, 0)),
        ],
        out_specs=[pl.BlockSpec((BM, HID), lambda i: (i, 0))] * 3,
        out_shape=[jax.ShapeDtypeStruct((N, HID), jnp.float32)] * 3,
    )(x, w0, dega, degb)


def _ln_relu(conv, g, b):
    mu = jnp.mean(conv, axis=1, keepdims=True)
    xc = conv - mu
    var = jnp.mean(xc * xc, axis=1, keepdims=True)
    h = xc * lax.rsqrt(var + 1e-5) * g + b
    return jnp.maximum(h, 0.0)


def _layer_body(acca_ref, accb_ref, hp_ref, dinv_ref, g_ref, bn_ref, bc_ref,
                wn_ref, h_ref, hpn_ref):
    dinv = dinv_ref[...]
    conv = dinv * (acca_ref[...] + accb_ref[...] + hp_ref[...]) + bc_ref[...]
    h = _ln_relu(conv, g_ref[...], bn_ref[...])
    h_ref[...] = h
    hpn_ref[...] = jnp.dot(h, wn_ref[...], preferred_element_type=jnp.float32) * dinv


def _tc_layer(acca, accb, hp, dinv, g, bn, bc, wn):
    return pl.pallas_call(
        _layer_body,
        grid=(GRID,),
        in_specs=[
            pl.BlockSpec((BM, HID), lambda i: (i, 0)),
            pl.BlockSpec((BM, HID), lambda i: (i, 0)),
            pl.BlockSpec((BM, HID), lambda i: (i, 0)),
            pl.BlockSpec((BM, HID), lambda i: (i, 0)),
            pl.BlockSpec((1, HID), lambda i: (0, 0)),
            pl.BlockSpec((1, HID), lambda i: (0, 0)),
            pl.BlockSpec((1, HID), lambda i: (0, 0)),
            pl.BlockSpec((HID, HID), lambda i: (0, 0)),
        ],
        out_specs=[pl.BlockSpec((BM, HID), lambda i: (i, 0))] * 2,
        out_shape=[jax.ShapeDtypeStruct((N, HID), jnp.float32)] * 2,
    )(acca, accb, hp, dinv, g, bn, bc, wn)


def _last_layer_body(acca_ref, accb_ref, hp_ref, dinv_ref, g_ref, bn_ref,
                     bc_ref, h_ref):
    conv = (dinv_ref[...] * (acca_ref[...] + accb_ref[...] + hp_ref[...])
            + bc_ref[...])
    h_ref[...] = _ln_relu(conv, g_ref[...], bn_ref[...])


def _tc_last_layer(acca, accb, hp, dinv, g, bn, bc):
    return pl.pallas_call(
        _last_layer_body,
        grid=(GRID,),
        in_specs=[
            pl.BlockSpec((BM, HID), lambda i: (i, 0)),
            pl.BlockSpec((BM, HID), lambda i: (i, 0)),
            pl.BlockSpec((BM, HID), lambda i: (i, 0)),
            pl.BlockSpec((BM, HID), lambda i: (i, 0)),
            pl.BlockSpec((1, HID), lambda i: (0, 0)),
            pl.BlockSpec((1, HID), lambda i: (0, 0)),
            pl.BlockSpec((1, HID), lambda i: (0, 0)),
        ],
        out_specs=pl.BlockSpec((BM, HID), lambda i: (i, 0)),
        out_shape=jax.ShapeDtypeStruct((N, HID), jnp.float32),
    )(acca, accb, hp, dinv, g, bn, bc)


def _final_body(h1_ref, h2_ref, h3_ref, h4_ref, wifT_ref, whfT_ref, bf_ref,
                wirT_ref, whrT_ref, br_ref, attw_ref, attb_ref, linw_ref,
                linb_ref, out_ref):
    s = [h1_ref[...], h2_ref[...], h3_ref[...], h4_ref[...]]

    def run_dir(order, wiT, whT, b):
        h = jnp.zeros((BM, LSTM_H), jnp.float32)
        c = jnp.zeros((BM, LSTM_H), jnp.float32)
        outs = {}
        for t in order:
            gates = (jnp.dot(s[t], wiT, preferred_element_type=jnp.float32)
                     + jnp.dot(h, whT, preferred_element_type=jnp.float32) + b)
            i_ = jax.nn.sigmoid(gates[:, 0 * LSTM_H:1 * LSTM_H])
            f_ = jax.nn.sigmoid(gates[:, 1 * LSTM_H:2 * LSTM_H])
            g_ = jnp.tanh(gates[:, 2 * LSTM_H:3 * LSTM_H])
            o_ = jax.nn.sigmoid(gates[:, 3 * LSTM_H:4 * LSTM_H])
            c = f_ * c + i_ * g_
            h = o_ * jnp.tanh(c)
            outs[t] = h
        return outs

    fwd = run_dir((0, 1, 2, 3), wifT_ref[...], whfT_ref[...], bf_ref[...])
    bwd = run_dir((3, 2, 1, 0), wirT_ref[...], whrT_ref[...], br_ref[...])

    awf = attw_ref[:, :LSTM_H]
    awb = attw_ref[:, LSTM_H:]
    attb = attb_ref[0, 0]
    logits = [
        jnp.sum(fwd[t] * awf, axis=1, keepdims=True)
        + jnp.sum(bwd[t] * awb, axis=1, keepdims=True) + attb
        for t in range(4)
    ]
    m = jnp.maximum(jnp.maximum(logits[0], logits[1]),
                    jnp.maximum(logits[2], logits[3]))
    es = [jnp.exp(l - m) for l in logits]
    z = es[0] + es[1] + es[2] + es[3]
    out16 = sum(es[t] * s[t] for t in range(4)) / z
    out_ref[...] = (jnp.dot(out16, linw_ref[...], preferred_element_type=jnp.float32)
                    + linb_ref[...])


def _tc_final(h1, h2, h3, h4, wifT, whfT, bf, wirT, whrT, br, attw, attb,
              linw, linb):
    blk = lambda r, c: pl.BlockSpec((r, c), lambda i: (0, 0))
    return pl.pallas_call(
        _final_body,
        grid=(GRID,),
        in_specs=[
            pl.BlockSpec((BM, HID), lambda i: (i, 0)),
            pl.BlockSpec((BM, HID), lambda i: (i, 0)),
            pl.BlockSpec((BM, HID), lambda i: (i, 0)),
            pl.BlockSpec((BM, HID), lambda i: (i, 0)),
            blk(HID, 4 * LSTM_H), blk(LSTM_H, 4 * LSTM_H), blk(1, 4 * LSTM_H),
            blk(HID, 4 * LSTM_H), blk(LSTM_H, 4 * LSTM_H), blk(1, 4 * LSTM_H),
            blk(1, 2 * LSTM_H), blk(1, 1), blk(HID, OUT_CH), blk(1, OUT_CH),
        ],
        out_specs=pl.BlockSpec((BM, OUT_CH), lambda i: (i, 0)),
        out_shape=jax.ShapeDtypeStruct((N, OUT_CH), jnp.float32),
    )(h1, h2, h3, h4, wifT, whfT, bf, wirT, whrT, br, attw, attb, linw, linb)


def kernel(x, edges, batch, w0, bc0, g0, bn0, w1, bc1, g1, bn1, w2, bc2, g2,
           bn2, w3, bc3, g3, bn3, lstm_wih, lstm_whh, lstm_bih, lstm_bhh,
           lstm_wih_r, lstm_whh_r, lstm_bih_r, lstm_bhh_r, att_w, att_b,
           lin_w, lin_b):
    src = edges[:, 0]
    dst = edges[:, 1]
    pad_dst = N + (jnp.arange(EPAD, dtype=jnp.int32) % DISCARD)
    srcr = jnp.concatenate([src, jnp.zeros((EPAD,), jnp.int32)]).reshape(RP, ROW)
    dstr = jnp.concatenate([dst, pad_dst]).reshape(RP, ROW)
    zero = jnp.zeros((NACC, HID), jnp.float32)
    ones = jnp.ones((ROW, HID), jnp.float32)

    def halves(o):
        return o[:N], o[NACC:NACC + N]

    dega, degb = halves(_sc_degree(dstr, ones, zero))

    h0, hp0, dinv = _tc0(x, w0, dega, degb)

    row = lambda v: v.reshape(1, -1)
    hs = []
    hp = hp0
    for (g, bn, bc, wn) in ((g0, bn0, bc0, w1), (g1, bn1, bc1, w2),
                            (g2, bn2, bc2, w3)):
        acca, accb = halves(_sc_gather_scatter(srcr, dstr, hp, zero))
        h, hp = _tc_layer(acca, accb, hp, dinv, row(g), row(bn), row(bc), wn)
        hs.append(h)
    acca, accb = halves(_sc_gather_scatter(srcr, dstr, hp, zero))
    hs.append(_tc_last_layer(acca, accb, hp, dinv, row(g3), row(bn3), row(bc3)))

    out = _tc_final(
        hs[0], hs[1], hs[2], hs[3],
        lstm_wih.T, lstm_whh.T, row(lstm_bih + lstm_bhh),
        lstm_wih_r.T, lstm_whh_r.T, row(lstm_bih_r + lstm_bhh_r),
        att_w, att_b.reshape(1, 1), lin_w, row(lin_b))
    return out


# speedup vs baseline: 25.8161x; 25.8161x over previous
import functools

import jax
import jax.numpy as jnp
from jax import lax
from jax.experimental import pallas as pl
from jax.experimental.pallas import tpu as pltpu
from jax.experimental.pallas import tpu_sc as plsc

N = 100000
E = 3200000
IN_CH = 896
HID = 16
OUT_CH = 640
LSTM_H = 32

NW = 32
K = 8
ROW = 128
RP = 25088
RW = RP // NW
EPAD = RP * ROW - E
DISCARD = 96
NACC = N + DISCARD
RPT = NACC // 16

_mesh = plsc.VectorSubcoreMesh(core_axis_name="c", subcore_axis_name="s")


@functools.partial(
    pl.kernel,
    out_type=jax.ShapeDtypeStruct((2 * NACC, HID), jnp.float32),
    mesh=_mesh,
    scratch_types=[
        pltpu.VMEM((K, ROW), jnp.int32),
        pltpu.VMEM((K, ROW), jnp.int32),
        pltpu.VMEM((K * ROW, HID), jnp.float32),
        pltpu.VMEM_SHARED((NACC, HID), jnp.float32),
        pltpu.SemaphoreType.DMA,
    ],
    compiler_params=pltpu.CompilerParams(use_tc_tiling_on_sc=False),
)
def _sc_gather_scatter(srcr, dstr, feat, zero, out, src_v, dst_v, rows_v, acc, sem):
    cid = lax.axis_index("c")
    sid = lax.axis_index("s")
    wid = sid * 2 + cid
    pltpu.sync_copy(zero.at[pl.ds(sid * RPT, RPT)], acc.at[pl.ds(sid * RPT, RPT)])
    plsc.subcore_barrier()
    base = wid * RW

    def body(g, carry):
        r0 = base + g * K
        pltpu.sync_copy(srcr.at[pl.ds(r0, K)], src_v)
        pltpu.sync_copy(dstr.at[pl.ds(r0, K)], dst_v)
        descs = []
        for j in range(K):
            descs.append(
                pltpu.async_copy(
                    feat.at[src_v.at[j]], rows_v.at[pl.ds(j * ROW, ROW)], sem
                )
            )
        for d in descs:
            d.wait()
        for j in range(K):
            pltpu.sync_copy(
                rows_v.at[pl.ds(j * ROW, ROW)], acc.at[dst_v.at[j]], add=True
            )
        return carry

    lax.fori_loop(0, RW // K, body, 0)
    plsc.subcore_barrier()
    pltpu.sync_copy(
        acc.at[pl.ds(sid * RPT, RPT)],
        out.at[pl.ds(cid * NACC + sid * RPT, RPT)],
    )


@functools.partial(
    pl.kernel,
    out_type=jax.ShapeDtypeStruct((2 * NACC, HID), jnp.float32),
    mesh=_mesh,
    scratch_types=[
        pltpu.VMEM((K, ROW), jnp.int32),
        pltpu.VMEM((ROW, HID), jnp.float32),
        pltpu.VMEM_SHARED((NACC, HID), jnp.float32),
    ],
    compiler_params=pltpu.CompilerParams(use_tc_tiling_on_sc=False),
)
def _sc_degree(dstr, ones, zero, out, dst_v, ones_v, acc):
    cid = lax.axis_index("c")
    sid = lax.axis_index("s")
    wid = sid * 2 + cid
    pltpu.sync_copy(zero.at[pl.ds(sid * RPT, RPT)], acc.at[pl.ds(sid * RPT, RPT)])
    pltpu.sync_copy(ones, ones_v)
    plsc.subcore_barrier()
    base = wid * RW

    def body(g, carry):
        r0 = base + g * K
        pltpu.sync_copy(dstr.at[pl.ds(r0, K)], dst_v)
        for j in range(K):
            pltpu.sync_copy(ones_v, acc.at[dst_v.at[j]], add=True)
        return carry

    lax.fori_loop(0, RW // K, body, 0)
    plsc.subcore_barrier()
    pltpu.sync_copy(
        acc.at[pl.ds(sid * RPT, RPT)],
        out.at[pl.ds(cid * NACC + sid * RPT, RPT)],
    )


BM = 2000
GRID = N // BM


def _tc0_body(x_ref, w0_ref, dega_ref, degb_ref, h0_ref, hp0_ref, dinv_ref):
    h0 = jnp.dot(x_ref[...], w0_ref[...], preferred_element_type=jnp.float32)
    deg = 1.0 + dega_ref[...] + degb_ref[...]
    dinv = lax.rsqrt(deg)
    h0_ref[...] = h0
    dinv_ref[...] = dinv
    hp0_ref[...] = h0 * dinv


def _tc0(x, w0, dega, degb):
    return pl.pallas_call(
        _tc0_body,
        grid=(GRID,),
        in_specs=[
            pl.BlockSpec((BM, IN_CH), lambda i: (i, 0)),
            pl.BlockSpec((IN_CH, HID), lambda i: (0, 0)),
            pl.BlockSpec((BM, HID), lambda i: (i, 0)),
            pl.BlockSpec((BM, HID), lambda i: (i, 0)),
        ],
        out_specs=[pl.BlockSpec((BM, HID), lambda i: (i, 0))] * 3,
        out_shape=[jax.ShapeDtypeStruct((N, HID), jnp.float32)] * 3,
    )(x, w0, dega, degb)


def _ln_relu(conv, g, b):
    mu = jnp.mean(conv, axis=1, keepdims=True)
    xc = conv - mu
    var = jnp.mean(xc * xc, axis=1, keepdims=True)
    h = xc * lax.rsqrt(var + 1e-5) * g + b
    return jnp.maximum(h, 0.0)


def _layer_body(acca_ref, accb_ref, hp_ref, dinv_ref, g_ref, bn_ref, bc_ref,
                wn_ref, h_ref, hpn_ref):
    dinv = dinv_ref[...]
    conv = dinv * (acca_ref[...] + accb_ref[...] + hp_ref[...]) + bc_ref[...]
    h = _ln_relu(conv, g_ref[...], bn_ref[...])
    h_ref[...] = h
    hpn_ref[...] = jnp.dot(h, wn_ref[...], preferred_element_type=jnp.float32) * dinv


def _tc_layer(acca, accb, hp, dinv, g, bn, bc, wn):
    return pl.pallas_call(
        _layer_body,
        grid=(GRID,),
        in_specs=[
            pl.BlockSpec((BM, HID), lambda i: (i, 0)),
            pl.BlockSpec((BM, HID), lambda i: (i, 0)),
            pl.BlockSpec((BM, HID), lambda i: (i, 0)),
            pl.BlockSpec((BM, HID), lambda i: (i, 0)),
            pl.BlockSpec((1, HID), lambda i: (0, 0)),
            pl.BlockSpec((1, HID), lambda i: (0, 0)),
            pl.BlockSpec((1, HID), lambda i: (0, 0)),
            pl.BlockSpec((HID, HID), lambda i: (0, 0)),
        ],
        out_specs=[pl.BlockSpec((BM, HID), lambda i: (i, 0))] * 2,
        out_shape=[jax.ShapeDtypeStruct((N, HID), jnp.float32)] * 2,
    )(acca, accb, hp, dinv, g, bn, bc, wn)


def _last_layer_body(acca_ref, accb_ref, hp_ref, dinv_ref, g_ref, bn_ref,
                     bc_ref, h_ref):
    conv = (dinv_ref[...] * (acca_ref[...] + accb_ref[...] + hp_ref[...])
            + bc_ref[...])
    h_ref[...] = _ln_relu(conv, g_ref[...], bn_ref[...])


def _tc_last_layer(acca, accb, hp, dinv, g, bn, bc):
    return pl.pallas_call(
        _last_layer_body,
        grid=(GRID,),
        in_specs=[
            pl.BlockSpec((BM, HID), lambda i: (i, 0)),
            pl.BlockSpec((BM, HID), lambda i: (i, 0)),
            pl.BlockSpec((BM, HID), lambda i: (i, 0)),
            pl.BlockSpec((BM, HID), lambda i: (i, 0)),
            pl.BlockSpec((1, HID), lambda i: (0, 0)),
            pl.BlockSpec((1, HID), lambda i: (0, 0)),
            pl.BlockSpec((1, HID), lambda i: (0, 0)),
        ],
        out_specs=pl.BlockSpec((BM, HID), lambda i: (i, 0)),
        out_shape=jax.ShapeDtypeStruct((N, HID), jnp.float32),
    )(acca, accb, hp, dinv, g, bn, bc)


def _final_body(h1_ref, h2_ref, h3_ref, h4_ref, wifT_ref, whfT_ref, bf_ref,
                wirT_ref, whrT_ref, br_ref, attw_ref, attb_ref, linw_ref,
                linb_ref, out_ref):
    s = [h1_ref[...], h2_ref[...], h3_ref[...], h4_ref[...]]

    def run_dir(order, wiT, whT, b):
        h = jnp.zeros((BM, LSTM_H), jnp.float32)
        c = jnp.zeros((BM, LSTM_H), jnp.float32)
        outs = {}
        for t in order:
            gates = (jnp.dot(s[t], wiT, preferred_element_type=jnp.float32)
                     + jnp.dot(h, whT, preferred_element_type=jnp.float32) + b)
            i_ = jax.nn.sigmoid(gates[:, 0 * LSTM_H:1 * LSTM_H])
            f_ = jax.nn.sigmoid(gates[:, 1 * LSTM_H:2 * LSTM_H])
            g_ = jnp.tanh(gates[:, 2 * LSTM_H:3 * LSTM_H])
            o_ = jax.nn.sigmoid(gates[:, 3 * LSTM_H:4 * LSTM_H])
            c = f_ * c + i_ * g_
            h = o_ * jnp.tanh(c)
            outs[t] = h
        return outs

    fwd = run_dir((0, 1, 2, 3), wifT_ref[...], whfT_ref[...], bf_ref[...])
    bwd = run_dir((3, 2, 1, 0), wirT_ref[...], whrT_ref[...], br_ref[...])

    awf = attw_ref[:, :LSTM_H]
    awb = attw_ref[:, LSTM_H:]
    attb = attb_ref[0, 0]
    logits = [
        jnp.sum(fwd[t] * awf, axis=1, keepdims=True)
        + jnp.sum(bwd[t] * awb, axis=1, keepdims=True) + attb
        for t in range(4)
    ]
    m = jnp.maximum(jnp.maximum(logits[0], logits[1]),
                    jnp.maximum(logits[2], logits[3]))
    es = [jnp.exp(l - m) for l in logits]
    z = es[0] + es[1] + es[2] + es[3]
    out16 = sum(es[t] * s[t] for t in range(4)) / z
    out_ref[...] = (jnp.dot(out16, linw_ref[...], preferred_element_type=jnp.float32)
                    + linb_ref[...])


def _tc_final(h1, h2, h3, h4, wifT, whfT, bf, wirT, whrT, br, attw, attb,
              linw, linb):
    blk = lambda r, c: pl.BlockSpec((r, c), lambda i: (0, 0))
    return pl.pallas_call(
        _final_body,
        grid=(GRID,),
        in_specs=[
            pl.BlockSpec((BM, HID), lambda i: (i, 0)),
            pl.BlockSpec((BM, HID), lambda i: (i, 0)),
            pl.BlockSpec((BM, HID), lambda i: (i, 0)),
            pl.BlockSpec((BM, HID), lambda i: (i, 0)),
            blk(HID, 4 * LSTM_H), blk(LSTM_H, 4 * LSTM_H), blk(1, 4 * LSTM_H),
            blk(HID, 4 * LSTM_H), blk(LSTM_H, 4 * LSTM_H), blk(1, 4 * LSTM_H),
            blk(1, 2 * LSTM_H), blk(1, 1), blk(HID, OUT_CH), blk(1, OUT_CH),
        ],
        out_specs=pl.BlockSpec((BM, OUT_CH), lambda i: (i, 0)),
        out_shape=jax.ShapeDtypeStruct((N, OUT_CH), jnp.float32),
    )(h1, h2, h3, h4, wifT, whfT, bf, wirT, whrT, br, attw, attb, linw, linb)


def kernel(x, edges, batch, w0, bc0, g0, bn0, w1, bc1, g1, bn1, w2, bc2, g2,
           bn2, w3, bc3, g3, bn3, lstm_wih, lstm_whh, lstm_bih, lstm_bhh,
           lstm_wih_r, lstm_whh_r, lstm_bih_r, lstm_bhh_r, att_w, att_b,
           lin_w, lin_b):
    src = edges[:, 0]
    dst = edges[:, 1]
    pad_dst = N + (jnp.arange(EPAD, dtype=jnp.int32) % DISCARD)
    srcr = jnp.concatenate([src, jnp.zeros((EPAD,), jnp.int32)]).reshape(RP, ROW)
    dstr = jnp.concatenate([dst, pad_dst]).reshape(RP, ROW)
    zero = jnp.zeros((NACC, HID), jnp.float32)
    ones = jnp.ones((ROW, HID), jnp.float32)

    def halves(o):
        return o[:N], o[NACC:NACC + N]

    dega, degb = halves(_sc_degree(dstr, ones, zero))

    h0, hp0, dinv = _tc0(x, w0, dega, degb)

    row = lambda v: v.reshape(1, -1)
    hs = []
    hp = hp0
    for (g, bn, bc, wn) in ((g0, bn0, bc0, w1), (g1, bn1, bc1, w2),
                            (g2, bn2, bc2, w3)):
        acca, accb = halves(_sc_gather_scatter(srcr, dstr, hp, zero))
        h, hp = _tc_layer(acca, accb, hp, dinv, row(g), row(bn), row(bc), wn)
        hs.append(h)
    acca, accb = halves(_sc_gather_scatter(srcr, dstr, hp, zero))
    hs.append(_tc_last_layer(acca, accb, hp, dinv, row(g3), row(bn3), row(bc3)))

    out = _tc_final(
        hs[0], hs[1], hs[2], hs[3],
        lstm_wih.T, lstm_whh.T, row(lstm_bih + lstm_bhh),
        lstm_wih_r.T, lstm_whh_r.T, row(lstm_bih_r + lstm_bhh_r),
        att_w, att_b.reshape(1, 1), lin_w, row(lin_b))
    return out


# speedup vs baseline: 29.9993x; 1.1620x over previous
import functools

import jax
import jax.numpy as jnp
from jax import lax
from jax.experimental import pallas as pl
from jax.experimental.pallas import tpu as pltpu
from jax.experimental.pallas import tpu_sc as plsc

N = 100000
E = 3200000
IN_CH = 896
HID = 16
OUT_CH = 640
LSTM_H = 32

NW = 32
EP = 3211264
EW = EP // NW
C = 512
NCH = EW // C
EPAD = EP - E
DISCARD = 96
NACC = N + DISCARD
RPT = NACC // 16

_mesh = plsc.VectorSubcoreMesh(core_axis_name="c", subcore_axis_name="s")


@functools.partial(
    pl.kernel,
    out_type=jax.ShapeDtypeStruct((2 * NACC, HID), jnp.float32),
    mesh=_mesh,
    scratch_types=[
        pltpu.VMEM((3, C), jnp.int32),
        pltpu.VMEM((3, C), jnp.int32),
        pltpu.VMEM((2, C, HID), jnp.float32),
        pltpu.VMEM_SHARED((NACC, HID), jnp.float32),
        pltpu.SemaphoreType.DMA,
        pltpu.SemaphoreType.DMA,
        pltpu.SemaphoreType.DMA,
    ],
    compiler_params=pltpu.CompilerParams(use_tc_tiling_on_sc=False),
)
def _sc_gather_scatter(srcf, dstf, feat, zero, out, idxs, idxd, rows, acc,
                       isem, gsem, ssem):
    cid = lax.axis_index("c")
    sid = lax.axis_index("s")
    wid = sid * 2 + cid
    pltpu.sync_copy(zero.at[pl.ds(sid * RPT, RPT)], acc.at[pl.ds(sid * RPT, RPT)])
    plsc.subcore_barrier()
    base = wid * EW

    def idx_descs(r, g):
        off = base + g * C
        return (
            pltpu.make_async_copy(srcf.at[pl.ds(off, C)], idxs.at[r], isem),
            pltpu.make_async_copy(dstf.at[pl.ds(off, C)], idxd.at[r], isem),
        )

    def gather_desc(r, p):
        return pltpu.make_async_copy(feat.at[idxs.at[r]], rows.at[p], gsem)

    def scatter_desc(r, p):
        return pltpu.make_async_copy(rows.at[p], acc.at[idxd.at[r]], ssem)

    for d in idx_descs(0, 0):
        d.start()
    for d in idx_descs(0, 0):
        d.wait()
    gather_desc(0, 0).start()
    for d in idx_descs(1, 1):
        d.start()

    def body(g, carry):
        p = lax.rem(g, 2)
        q = 1 - p
        r = lax.rem(g, 3)
        r1 = lax.rem(g + 1, 3)
        r2 = lax.rem(g + 2, 3)

        gather_desc(r, p).wait()

        @pl.when(g >= 1)
        def _():
            scatter_desc(r2, q).wait()

        scatter_desc(r, p).start(add=True)

        @pl.when(g <= NCH - 2)
        def _():
            for d in idx_descs(r1, g + 1):
                d.wait()
            gather_desc(r1, q).start()

        @pl.when(g <= NCH - 3)
        def _():
            for d in idx_descs(r2, g + 2):
                d.start()
        return carry

    lax.fori_loop(0, NCH, body, 0)
    scatter_desc((NCH - 1) % 3, (NCH - 1) % 2).wait()
    plsc.subcore_barrier()
    pltpu.sync_copy(
        acc.at[pl.ds(sid * RPT, RPT)],
        out.at[pl.ds(cid * NACC + sid * RPT, RPT)],
    )


@functools.partial(
    pl.kernel,
    out_type=jax.ShapeDtypeStruct((2 * NACC, HID), jnp.float32),
    mesh=_mesh,
    scratch_types=[
        pltpu.VMEM((3, C), jnp.int32),
        pltpu.VMEM((C, HID), jnp.float32),
        pltpu.VMEM_SHARED((NACC, HID), jnp.float32),
        pltpu.SemaphoreType.DMA,
        pltpu.SemaphoreType.DMA,
    ],
    compiler_params=pltpu.CompilerParams(use_tc_tiling_on_sc=False),
)
def _sc_degree(dstf, ones, zero, out, idxd, ones_v, acc, isem, ssem):
    cid = lax.axis_index("c")
    sid = lax.axis_index("s")
    wid = sid * 2 + cid
    pltpu.sync_copy(zero.at[pl.ds(sid * RPT, RPT)], acc.at[pl.ds(sid * RPT, RPT)])
    pltpu.sync_copy(ones, ones_v)
    plsc.subcore_barrier()
    base = wid * EW

    def idx_desc(r, g):
        return pltpu.make_async_copy(dstf.at[pl.ds(base + g * C, C)],
                                     idxd.at[r], isem)

    def scatter_desc(r):
        return pltpu.make_async_copy(ones_v, acc.at[idxd.at[r]], ssem)

    idx_desc(0, 0).start()
    idx_desc(0, 0).wait()
    idx_desc(1, 1).start()

    def body(g, carry):
        r = lax.rem(g, 3)
        r1 = lax.rem(g + 1, 3)
        r2 = lax.rem(g + 2, 3)

        @pl.when(g >= 1)
        def _():
            scatter_desc(r2).wait()

        scatter_desc(r).start(add=True)

        @pl.when(g <= NCH - 2)
        def _():
            idx_desc(r1, g + 1).wait()

        @pl.when(g <= NCH - 3)
        def _():
            idx_desc(r2, g + 2).start()
        return carry

    lax.fori_loop(0, NCH, body, 0)
    scatter_desc((NCH - 1) % 3).wait()
    plsc.subcore_barrier()
    pltpu.sync_copy(
        acc.at[pl.ds(sid * RPT, RPT)],
        out.at[pl.ds(cid * NACC + sid * RPT, RPT)],
    )


BM = 2000
GRID = N // BM


def _tc0_body(x_ref, w0_ref, dega_ref, degb_ref, h0_ref, hp0_ref, dinv_ref):
    h0 = jnp.dot(x_ref[...], w0_ref[...], preferred_element_type=jnp.float32)
    deg = 1.0 + dega_ref[...] + degb_ref[...]
    dinv = lax.rsqrt(deg)
    h0_ref[...] = h0
    dinv_ref[...] = dinv
    hp0_ref[...] = h0 * dinv


def _tc0(x, w0, dega, degb):
    return pl.pallas_call(
        _tc0_body,
        grid=(GRID,),
        in_specs=[
            pl.BlockSpec((BM, IN_CH), lambda i: (i, 0)),
            pl.BlockSpec((IN_CH, HID), lambda i: (0, 0)),
            pl.BlockSpec((BM, HID), lambda i: (i, 0)),
            pl.BlockSpec((BM, HID), lambda i: (i, 0)),
        ],
        out_specs=[pl.BlockSpec((BM, HID), lambda i: (i, 0))] * 3,
        out_shape=[jax.ShapeDtypeStruct((N, HID), jnp.float32)] * 3,
    )(x, w0, dega, degb)


def _ln_relu(conv, g, b):
    mu = jnp.mean(conv, axis=1, keepdims=True)
    xc = conv - mu
    var = jnp.mean(xc * xc, axis=1, keepdims=True)
    h = xc * lax.rsqrt(var + 1e-5) * g + b
    return jnp.maximum(h, 0.0)


def _layer_body(acca_ref, accb_ref, hp_ref, dinv_ref, g_ref, bn_ref, bc_ref,
                wn_ref, h_ref, hpn_ref):
    dinv = dinv_ref[...]
    conv = dinv * (acca_ref[...] + accb_ref[...] + hp_ref[...]) + bc_ref[...]
    h = _ln_relu(conv, g_ref[...], bn_ref[...])
    h_ref[...] = h
    hpn_ref[...] = jnp.dot(h, wn_ref[...], preferred_element_type=jnp.float32) * dinv


def _tc_layer(acca, accb, hp, dinv, g, bn, bc, wn):
    return pl.pallas_call(
        _layer_body,
        grid=(GRID,),
        in_specs=[
            pl.BlockSpec((BM, HID), lambda i: (i, 0)),
            pl.BlockSpec((BM, HID), lambda i: (i, 0)),
            pl.BlockSpec((BM, HID), lambda i: (i, 0)),
            pl.BlockSpec((BM, HID), lambda i: (i, 0)),
            pl.BlockSpec((1, HID), lambda i: (0, 0)),
            pl.BlockSpec((1, HID), lambda i: (0, 0)),
            pl.BlockSpec((1, HID), lambda i: (0, 0)),
            pl.BlockSpec((HID, HID), lambda i: (0, 0)),
        ],
        out_specs=[pl.BlockSpec((BM, HID), lambda i: (i, 0))] * 2,
        out_shape=[jax.ShapeDtypeStruct((N, HID), jnp.float32)] * 2,
    )(acca, accb, hp, dinv, g, bn, bc, wn)


def _last_layer_body(acca_ref, accb_ref, hp_ref, dinv_ref, g_ref, bn_ref,
                     bc_ref, h_ref):
    conv = (dinv_ref[...] * (acca_ref[...] + accb_ref[...] + hp_ref[...])
            + bc_ref[...])
    h_ref[...] = _ln_relu(conv, g_ref[...], bn_ref[...])


def _tc_last_layer(acca, accb, hp, dinv, g, bn, bc):
    return pl.pallas_call(
        _last_layer_body,
        grid=(GRID,),
        in_specs=[
            pl.BlockSpec((BM, HID), lambda i: (i, 0)),
            pl.BlockSpec((BM, HID), lambda i: (i, 0)),
            pl.BlockSpec((BM, HID), lambda i: (i, 0)),
            pl.BlockSpec((BM, HID), lambda i: (i, 0)),
            pl.BlockSpec((1, HID), lambda i: (0, 0)),
            pl.BlockSpec((1, HID), lambda i: (0, 0)),
            pl.BlockSpec((1, HID), lambda i: (0, 0)),
        ],
        out_specs=pl.BlockSpec((BM, HID), lambda i: (i, 0)),
        out_shape=jax.ShapeDtypeStruct((N, HID), jnp.float32),
    )(acca, accb, hp, dinv, g, bn, bc)


def _final_body(h1_ref, h2_ref, h3_ref, h4_ref, wifT_ref, whfT_ref, bf_ref,
                wirT_ref, whrT_ref, br_ref, attw_ref, attb_ref, linw_ref,
                linb_ref, out_ref):
    s = [h1_ref[...], h2_ref[...], h3_ref[...], h4_ref[...]]

    def run_dir(order, wiT, whT, b):
        h = jnp.zeros((BM, LSTM_H), jnp.float32)
        c = jnp.zeros((BM, LSTM_H), jnp.float32)
        outs = {}
        for t in order:
            gates = (jnp.dot(s[t], wiT, preferred_element_type=jnp.float32)
                     + jnp.dot(h, whT, preferred_element_type=jnp.float32) + b)
            i_ = jax.nn.sigmoid(gates[:, 0 * LSTM_H:1 * LSTM_H])
            f_ = jax.nn.sigmoid(gates[:, 1 * LSTM_H:2 * LSTM_H])
            g_ = jnp.tanh(gates[:, 2 * LSTM_H:3 * LSTM_H])
            o_ = jax.nn.sigmoid(gates[:, 3 * LSTM_H:4 * LSTM_H])
            c = f_ * c + i_ * g_
            h = o_ * jnp.tanh(c)
            outs[t] = h
        return outs

    fwd = run_dir((0, 1, 2, 3), wifT_ref[...], whfT_ref[...], bf_ref[...])
    bwd = run_dir((3, 2, 1, 0), wirT_ref[...], whrT_ref[...], br_ref[...])

    awf = attw_ref[:, :LSTM_H]
    awb = attw_ref[:, LSTM_H:]
    attb = attb_ref[0, 0]
    logits = [
        jnp.sum(fwd[t] * awf, axis=1, keepdims=True)
        + jnp.sum(bwd[t] * awb, axis=1, keepdims=True) + attb
        for t in range(4)
    ]
    m = jnp.maximum(jnp.maximum(logits[0], logits[1]),
                    jnp.maximum(logits[2], logits[3]))
    es = [jnp.exp(l - m) for l in logits]
    z = es[0] + es[1] + es[2] + es[3]
    out16 = sum(es[t] * s[t] for t in range(4)) / z
    out_ref[...] = (jnp.dot(out16, linw_ref[...], preferred_element_type=jnp.float32)
                    + linb_ref[...])


def _tc_final(h1, h2, h3, h4, wifT, whfT, bf, wirT, whrT, br, attw, attb,
              linw, linb):
    blk = lambda r, c: pl.BlockSpec((r, c), lambda i: (0, 0))
    return pl.pallas_call(
        _final_body,
        grid=(GRID,),
        in_specs=[
            pl.BlockSpec((BM, HID), lambda i: (i, 0)),
            pl.BlockSpec((BM, HID), lambda i: (i, 0)),
            pl.BlockSpec((BM, HID), lambda i: (i, 0)),
            pl.BlockSpec((BM, HID), lambda i: (i, 0)),
            blk(HID, 4 * LSTM_H), blk(LSTM_H, 4 * LSTM_H), blk(1, 4 * LSTM_H),
            blk(HID, 4 * LSTM_H), blk(LSTM_H, 4 * LSTM_H), blk(1, 4 * LSTM_H),
            blk(1, 2 * LSTM_H), blk(1, 1), blk(HID, OUT_CH), blk(1, OUT_CH),
        ],
        out_specs=pl.BlockSpec((BM, OUT_CH), lambda i: (i, 0)),
        out_shape=jax.ShapeDtypeStruct((N, OUT_CH), jnp.float32),
    )(h1, h2, h3, h4, wifT, whfT, bf, wirT, whrT, br, attw, attb, linw, linb)


def kernel(x, edges, batch, w0, bc0, g0, bn0, w1, bc1, g1, bn1, w2, bc2, g2,
           bn2, w3, bc3, g3, bn3, lstm_wih, lstm_whh, lstm_bih, lstm_bhh,
           lstm_wih_r, lstm_whh_r, lstm_bih_r, lstm_bhh_r, att_w, att_b,
           lin_w, lin_b):
    src = edges[:, 0]
    dst = edges[:, 1]
    pad_dst = N + (jnp.arange(EPAD, dtype=jnp.int32) % DISCARD)
    srcf = jnp.concatenate([src, jnp.zeros((EPAD,), jnp.int32)])
    dstf = jnp.concatenate([dst, pad_dst])
    zero = jnp.zeros((NACC, HID), jnp.float32)
    ones = jnp.ones((C, HID), jnp.float32)

    def halves(o):
        return o[:N], o[NACC:NACC + N]

    dega, degb = halves(_sc_degree(dstf, ones, zero))

    h0, hp0, dinv = _tc0(x, w0, dega, degb)

    row = lambda v: v.reshape(1, -1)
    hs = []
    hp = hp0
    for (g, bn, bc, wn) in ((g0, bn0, bc0, w1), (g1, bn1, bc1, w2),
                            (g2, bn2, bc2, w3)):
        acca, accb = halves(_sc_gather_scatter(srcf, dstf, hp, zero))
        h, hp = _tc_layer(acca, accb, hp, dinv, row(g), row(bn), row(bc), wn)
        hs.append(h)
    acca, accb = halves(_sc_gather_scatter(srcf, dstf, hp, zero))
    hs.append(_tc_last_layer(acca, accb, hp, dinv, row(g3), row(bn3), row(bc3)))

    out = _tc_final(
        hs[0], hs[1], hs[2], hs[3],
        lstm_wih.T, lstm_whh.T, row(lstm_bih + lstm_bhh),
        lstm_wih_r.T, lstm_whh_r.T, row(lstm_bih_r + lstm_bhh_r),
        att_w, att_b.reshape(1, 1), lin_w, row(lin_b))
    return out


# speedup vs baseline: 33.1105x; 1.1037x over previous
import functools

import jax
import jax.numpy as jnp
from jax import lax
from jax.experimental import pallas as pl
from jax.experimental.pallas import tpu as pltpu
from jax.experimental.pallas import tpu_sc as plsc

N = 100000
E = 3200000
IN_CH = 896
HID = 16
OUT_CH = 640
LSTM_H = 32

NW = 32
EP = 3211264
EW = EP // NW
C = 512
NCH = EW // C
EPAD = EP - E
DISCARD = 96
NACC = N + DISCARD
RPT = NACC // 16

_mesh = plsc.VectorSubcoreMesh(core_axis_name="c", subcore_axis_name="s",
                               num_cores=2, num_subcores=16)


@functools.partial(
    pl.kernel,
    out_type=jax.ShapeDtypeStruct((2 * NACC, HID), jnp.float32),
    mesh=_mesh,
    scratch_types=[
        pltpu.VMEM((3, C), jnp.int32),
        pltpu.VMEM((3, C), jnp.int32),
        pltpu.VMEM((2, C, HID), jnp.float32),
        pltpu.VMEM_SHARED((NACC, HID), jnp.float32),
        pltpu.SemaphoreType.DMA,
        pltpu.SemaphoreType.DMA,
        pltpu.SemaphoreType.DMA,
    ],
    compiler_params=pltpu.CompilerParams(use_tc_tiling_on_sc=False),
)
def _sc_gather_scatter(srcf, dstf, feat, zero, out, idxs, idxd, rows, acc,
                       isem, gsem, ssem):
    cid = lax.axis_index("c")
    sid = lax.axis_index("s")
    wid = sid * 2 + cid
    pltpu.sync_copy(zero.at[pl.ds(sid * RPT, RPT)], acc.at[pl.ds(sid * RPT, RPT)])
    plsc.subcore_barrier()
    base = wid * EW

    def idx_descs(r, g):
        off = base + g * C
        return (
            pltpu.make_async_copy(srcf.at[pl.ds(off, C)], idxs.at[r], isem),
            pltpu.make_async_copy(dstf.at[pl.ds(off, C)], idxd.at[r], isem),
        )

    def gather_desc(r, p):
        return pltpu.make_async_copy(feat.at[idxs.at[r]], rows.at[p], gsem)

    def scatter_desc(r, p):
        return pltpu.make_async_copy(rows.at[p], acc.at[idxd.at[r]], ssem)

    for d in idx_descs(0, 0):
        d.start()
    for d in idx_descs(0, 0):
        d.wait()
    gather_desc(0, 0).start()
    for d in idx_descs(1, 1):
        d.start()

    def body(g, carry):
        p = lax.rem(g, 2)
        q = 1 - p
        r = lax.rem(g, 3)
        r1 = lax.rem(g + 1, 3)
        r2 = lax.rem(g + 2, 3)

        gather_desc(r, p).wait()

        @pl.when(g >= 1)
        def _():
            scatter_desc(r2, q).wait()

        scatter_desc(r, p).start(add=True)

        @pl.when(g <= NCH - 2)
        def _():
            for d in idx_descs(r1, g + 1):
                d.wait()
            gather_desc(r1, q).start()

        @pl.when(g <= NCH - 3)
        def _():
            for d in idx_descs(r2, g + 2):
                d.start()
        return carry

    lax.fori_loop(0, NCH, body, 0)
    scatter_desc((NCH - 1) % 3, (NCH - 1) % 2).wait()
    plsc.subcore_barrier()
    pltpu.sync_copy(
        acc.at[pl.ds(sid * RPT, RPT)],
        out.at[pl.ds(cid * NACC + sid * RPT, RPT)],
    )


@functools.partial(
    pl.kernel,
    out_type=jax.ShapeDtypeStruct((2 * NACC, HID), jnp.float32),
    mesh=_mesh,
    scratch_types=[
        pltpu.VMEM((3, C), jnp.int32),
        pltpu.VMEM((C, HID), jnp.float32),
        pltpu.VMEM_SHARED((NACC, HID), jnp.float32),
        pltpu.SemaphoreType.DMA,
        pltpu.SemaphoreType.DMA,
    ],
    compiler_params=pltpu.CompilerParams(use_tc_tiling_on_sc=False),
)
def _sc_degree(dstf, ones, zero, out, idxd, ones_v, acc, isem, ssem):
    cid = lax.axis_index("c")
    sid = lax.axis_index("s")
    wid = sid * 2 + cid
    pltpu.sync_copy(zero.at[pl.ds(sid * RPT, RPT)], acc.at[pl.ds(sid * RPT, RPT)])
    pltpu.sync_copy(ones, ones_v)
    plsc.subcore_barrier()
    base = wid * EW

    def idx_desc(r, g):
        return pltpu.make_async_copy(dstf.at[pl.ds(base + g * C, C)],
                                     idxd.at[r], isem)

    def scatter_desc(r):
        return pltpu.make_async_copy(ones_v, acc.at[idxd.at[r]], ssem)

    idx_desc(0, 0).start()
    idx_desc(0, 0).wait()
    idx_desc(1, 1).start()

    def body(g, carry):
        r = lax.rem(g, 3)
        r1 = lax.rem(g + 1, 3)
        r2 = lax.rem(g + 2, 3)

        @pl.when(g >= 1)
        def _():
            scatter_desc(r2).wait()

        scatter_desc(r).start(add=True)

        @pl.when(g <= NCH - 2)
        def _():
            idx_desc(r1, g + 1).wait()

        @pl.when(g <= NCH - 3)
        def _():
            idx_desc(r2, g + 2).start()
        return carry

    lax.fori_loop(0, NCH, body, 0)
    scatter_desc((NCH - 1) % 3).wait()
    plsc.subcore_barrier()
    pltpu.sync_copy(
        acc.at[pl.ds(sid * RPT, RPT)],
        out.at[pl.ds(cid * NACC + sid * RPT, RPT)],
    )


BM = 2000
GRID = N // BM


def _tc0_body(x_ref, w0_ref, dega_ref, degb_ref, h0_ref, hp0_ref, dinv_ref):
    h0 = jnp.dot(x_ref[...], w0_ref[...], preferred_element_type=jnp.float32)
    deg = 1.0 + dega_ref[...] + degb_ref[...]
    dinv = lax.rsqrt(deg)
    h0_ref[...] = h0
    dinv_ref[...] = dinv
    hp0_ref[...] = h0 * dinv


def _tc0(x, w0, dega, degb):
    return pl.pallas_call(
        _tc0_body,
        grid=(GRID,),
        in_specs=[
            pl.BlockSpec((BM, IN_CH), lambda i: (i, 0)),
            pl.BlockSpec((IN_CH, HID), lambda i: (0, 0)),
            pl.BlockSpec((BM, HID), lambda i: (i, 0)),
            pl.BlockSpec((BM, HID), lambda i: (i, 0)),
        ],
        out_specs=[pl.BlockSpec((BM, HID), lambda i: (i, 0))] * 3,
        out_shape=[jax.ShapeDtypeStruct((N, HID), jnp.float32)] * 3,
    )(x, w0, dega, degb)


def _ln_relu(conv, g, b):
    mu = jnp.mean(conv, axis=1, keepdims=True)
    xc = conv - mu
    var = jnp.mean(xc * xc, axis=1, keepdims=True)
    h = xc * lax.rsqrt(var + 1e-5) * g + b
    return jnp.maximum(h, 0.0)


def _layer_body(acca_ref, accb_ref, hp_ref, dinv_ref, g_ref, bn_ref, bc_ref,
                wn_ref, h_ref, hpn_ref):
    dinv = dinv_ref[...]
    conv = dinv * (acca_ref[...] + accb_ref[...] + hp_ref[...]) + bc_ref[...]
    h = _ln_relu(conv, g_ref[...], bn_ref[...])
    h_ref[...] = h
    hpn_ref[...] = jnp.dot(h, wn_ref[...], preferred_element_type=jnp.float32) * dinv


def _tc_layer(acca, accb, hp, dinv, g, bn, bc, wn):
    return pl.pallas_call(
        _layer_body,
        grid=(GRID,),
        in_specs=[
            pl.BlockSpec((BM, HID), lambda i: (i, 0)),
            pl.BlockSpec((BM, HID), lambda i: (i, 0)),
            pl.BlockSpec((BM, HID), lambda i: (i, 0)),
            pl.BlockSpec((BM, HID), lambda i: (i, 0)),
            pl.BlockSpec((1, HID), lambda i: (0, 0)),
            pl.BlockSpec((1, HID), lambda i: (0, 0)),
            pl.BlockSpec((1, HID), lambda i: (0, 0)),
            pl.BlockSpec((HID, HID), lambda i: (0, 0)),
        ],
        out_specs=[pl.BlockSpec((BM, HID), lambda i: (i, 0))] * 2,
        out_shape=[jax.ShapeDtypeStruct((N, HID), jnp.float32)] * 2,
    )(acca, accb, hp, dinv, g, bn, bc, wn)


def _last_layer_body(acca_ref, accb_ref, hp_ref, dinv_ref, g_ref, bn_ref,
                     bc_ref, h_ref):
    conv = (dinv_ref[...] * (acca_ref[...] + accb_ref[...] + hp_ref[...])
            + bc_ref[...])
    h_ref[...] = _ln_relu(conv, g_ref[...], bn_ref[...])


def _tc_last_layer(acca, accb, hp, dinv, g, bn, bc):
    return pl.pallas_call(
        _last_layer_body,
        grid=(GRID,),
        in_specs=[
            pl.BlockSpec((BM, HID), lambda i: (i, 0)),
            pl.BlockSpec((BM, HID), lambda i: (i, 0)),
            pl.BlockSpec((BM, HID), lambda i: (i, 0)),
            pl.BlockSpec((BM, HID), lambda i: (i, 0)),
            pl.BlockSpec((1, HID), lambda i: (0, 0)),
            pl.BlockSpec((1, HID), lambda i: (0, 0)),
            pl.BlockSpec((1, HID), lambda i: (0, 0)),
        ],
        out_specs=pl.BlockSpec((BM, HID), lambda i: (i, 0)),
        out_shape=jax.ShapeDtypeStruct((N, HID), jnp.float32),
    )(acca, accb, hp, dinv, g, bn, bc)


def _final_body(h1_ref, h2_ref, h3_ref, h4_ref, w2_ref, b2_ref, aw_ref,
                attb_ref, linw_ref, linb_ref, out_ref):
    s = [h1_ref[...], h2_ref[...], h3_ref[...], h4_ref[...]]
    w2 = w2_ref[...]
    b2 = b2_ref[...]
    H = LSTM_H
    hf = jnp.zeros((BM, H), jnp.float32)
    hb = jnp.zeros((BM, H), jnp.float32)
    cf = jnp.zeros((BM, H), jnp.float32)
    cb = jnp.zeros((BM, H), jnp.float32)
    fwd = [None] * 4
    bwd = [None] * 4
    for t in range(4):
        xt = jnp.concatenate([s[t], hf, s[3 - t], hb], axis=1)
        gates = jnp.dot(xt, w2, preferred_element_type=jnp.float32) + b2
        sg = jax.nn.sigmoid(gates[:, :192])
        tg = jnp.tanh(gates[:, 192:])
        cf = sg[:, 32:64] * cf + sg[:, 0:32] * tg[:, 0:32]
        cb = sg[:, 128:160] * cb + sg[:, 96:128] * tg[:, 32:64]
        tc2 = jnp.tanh(jnp.concatenate([cf, cb], axis=1))
        hf = sg[:, 64:96] * tc2[:, :H]
        hb = sg[:, 160:192] * tc2[:, H:]
        fwd[t] = hf
        bwd[3 - t] = hb
    lcat = jnp.concatenate(
        [jnp.concatenate([fwd[t], bwd[t]], axis=1) for t in range(4)], axis=1)
    logits = (jnp.dot(lcat, aw_ref[...], preferred_element_type=jnp.float32)
              + attb_ref[0, 0])
    m = jnp.max(logits, axis=1, keepdims=True)
    e = jnp.exp(logits - m)
    z = jnp.sum(e, axis=1, keepdims=True)
    out16 = sum(e[:, t:t + 1] * s[t] for t in range(4)) / z
    out_ref[...] = (jnp.dot(out16, linw_ref[...], preferred_element_type=jnp.float32)
                    + linb_ref[...])


def _tc_final(h1, h2, h3, h4, w2, b2, aw, attb, linw, linb):
    blk = lambda r, c: pl.BlockSpec((r, c), lambda i: (0, 0))
    return pl.pallas_call(
        _final_body,
        grid=(GRID,),
        in_specs=[
            pl.BlockSpec((BM, HID), lambda i: (i, 0)),
            pl.BlockSpec((BM, HID), lambda i: (i, 0)),
            pl.BlockSpec((BM, HID), lambda i: (i, 0)),
            pl.BlockSpec((BM, HID), lambda i: (i, 0)),
            blk(96, 256), blk(1, 256), blk(256, 4), blk(1, 1),
            blk(HID, OUT_CH), blk(1, OUT_CH),
        ],
        out_specs=pl.BlockSpec((BM, OUT_CH), lambda i: (i, 0)),
        out_shape=jax.ShapeDtypeStruct((N, OUT_CH), jnp.float32),
    )(h1, h2, h3, h4, w2, b2, aw, attb, linw, linb)


def kernel(x, edges, batch, w0, bc0, g0, bn0, w1, bc1, g1, bn1, w2, bc2, g2,
           bn2, w3, bc3, g3, bn3, lstm_wih, lstm_whh, lstm_bih, lstm_bhh,
           lstm_wih_r, lstm_whh_r, lstm_bih_r, lstm_bhh_r, att_w, att_b,
           lin_w, lin_b):
    src = edges[:, 0]
    dst = edges[:, 1]
    pad_dst = N + (jnp.arange(EPAD, dtype=jnp.int32) % DISCARD)
    srcf = jnp.concatenate([src, jnp.zeros((EPAD,), jnp.int32)])
    dstf = jnp.concatenate([dst, pad_dst])
    zero = jnp.zeros((NACC, HID), jnp.float32)
    ones = jnp.ones((C, HID), jnp.float32)

    def halves(o):
        return o[:N], o[NACC:NACC + N]

    dega, degb = halves(_sc_degree(dstf, ones, zero))

    h0, hp0, dinv = _tc0(x, w0, dega, degb)

    row = lambda v: v.reshape(1, -1)
    hs = []
    hp = hp0
    for (g, bn, bc, wn) in ((g0, bn0, bc0, w1), (g1, bn1, bc1, w2),
                            (g2, bn2, bc2, w3)):
        acca, accb = halves(_sc_gather_scatter(srcf, dstf, hp, zero))
        h, hp = _tc_layer(acca, accb, hp, dinv, row(g), row(bn), row(bc), wn)
        hs.append(h)
    acca, accb = halves(_sc_gather_scatter(srcf, dstf, hp, zero))
    hs.append(_tc_last_layer(acca, accb, hp, dinv, row(g3), row(bn3), row(bc3)))

    def lstm_pack(wih, whh, bih, bhh):
        wd = jnp.concatenate([wih.T, whh.T], axis=0)
        sig = jnp.concatenate([wd[:, 0:64], wd[:, 96:128]], axis=1)
        b = bih + bhh
        bsig = jnp.concatenate([b[0:64], b[96:128]])
        return sig, wd[:, 64:96], bsig, b[64:96]

    sigf, gf, bsf, bgf = lstm_pack(lstm_wih, lstm_whh, lstm_bih, lstm_bhh)
    sigb, gb, bsb, bgb = lstm_pack(lstm_wih_r, lstm_whh_r, lstm_bih_r,
                                   lstm_bhh_r)
    z96 = jnp.zeros((48, 96), jnp.float32)
    z32 = jnp.zeros((48, 32), jnp.float32)
    w2 = jnp.concatenate([
        jnp.concatenate([sigf, z96, gf, z32], axis=1),
        jnp.concatenate([z96, sigb, z32, gb], axis=1),
    ], axis=0)
    b2 = jnp.concatenate([bsf, bsb, bgf, bgb]).reshape(1, 256)
    aw = jnp.kron(jnp.eye(4, dtype=jnp.float32), att_w[0][:, None])

    out = _tc_final(hs[0], hs[1], hs[2], hs[3], w2, b2, aw,
                    att_b.reshape(1, 1), lin_w, row(lin_b))
    return out


# speedup vs baseline: 37.9399x; 1.1459x over previous
import functools

import jax
import jax.numpy as jnp
from jax import lax
from jax.experimental import pallas as pl
from jax.experimental.pallas import tpu as pltpu
from jax.experimental.pallas import tpu_sc as plsc

N = 100000
E = 3200000
IN_CH = 896
HID = 16
OUT_CH = 640
LSTM_H = 32

NW = 32
EP = 3211264
EW = EP // NW
C = 512
NCH = EW // C
EPAD = EP - E
DISCARD = 352
NACC = N + DISCARD
RPT = NACC // 16

_mesh = plsc.VectorSubcoreMesh(core_axis_name="c", subcore_axis_name="s",
                               num_cores=2, num_subcores=16)


@functools.partial(
    pl.kernel,
    out_type=jax.ShapeDtypeStruct((2 * NACC, HID), jnp.float32),
    mesh=_mesh,
    scratch_types=[
        pltpu.VMEM((3, C), jnp.int32),
        pltpu.VMEM((3, C), jnp.int32),
        pltpu.VMEM((2, C, HID), jnp.float32),
        pltpu.VMEM_SHARED((NACC, HID), jnp.float32),
        pltpu.SemaphoreType.DMA,
        pltpu.SemaphoreType.DMA,
        pltpu.SemaphoreType.DMA,
    ],
    compiler_params=pltpu.CompilerParams(use_tc_tiling_on_sc=False),
)
def _sc_gather_scatter(srcf, dstf, feat, zero, out, idxs, idxd, rows, acc,
                       isem, gsem, ssem):
    cid = lax.axis_index("c")
    sid = lax.axis_index("s")
    wid = sid * 2 + cid
    pltpu.sync_copy(zero.at[pl.ds(sid * RPT, RPT)], acc.at[pl.ds(sid * RPT, RPT)])
    plsc.subcore_barrier()
    base = wid * EW

    def idx_descs(r, g):
        off = base + g * C
        return (
            pltpu.make_async_copy(srcf.at[pl.ds(off, C)], idxs.at[r], isem),
            pltpu.make_async_copy(dstf.at[pl.ds(off, C)], idxd.at[r], isem),
        )

    def gather_desc(r, p):
        return pltpu.make_async_copy(feat.at[idxs.at[r]], rows.at[p], gsem)

    def scatter_desc(r, p):
        return pltpu.make_async_copy(rows.at[p], acc.at[idxd.at[r]], ssem)

    for d in idx_descs(0, 0):
        d.start()
    for d in idx_descs(0, 0):
        d.wait()
    gather_desc(0, 0).start()
    for d in idx_descs(1, 1):
        d.start()

    def body(g, carry):
        p = lax.rem(g, 2)
        q = 1 - p
        r = lax.rem(g, 3)
        r1 = lax.rem(g + 1, 3)
        r2 = lax.rem(g + 2, 3)

        gather_desc(r, p).wait()

        @pl.when(g >= 1)
        def _():
            scatter_desc(r2, q).wait()

        scatter_desc(r, p).start(add=True)

        @pl.when(g <= NCH - 2)
        def _():
            for d in idx_descs(r1, g + 1):
                d.wait()
            gather_desc(r1, q).start()

        @pl.when(g <= NCH - 3)
        def _():
            for d in idx_descs(r2, g + 2):
                d.start()
        return carry

    lax.fori_loop(0, NCH, body, 0)
    scatter_desc((NCH - 1) % 3, (NCH - 1) % 2).wait()
    plsc.subcore_barrier()
    pltpu.sync_copy(
        acc.at[pl.ds(sid * RPT, RPT)],
        out.at[pl.ds(cid * NACC + sid * RPT, RPT)],
    )


@functools.partial(
    pl.kernel,
    out_type=jax.ShapeDtypeStruct((2 * NACC, HID), jnp.float32),
    mesh=_mesh,
    scratch_types=[
        pltpu.VMEM((3, C), jnp.int32),
        pltpu.VMEM((C, HID), jnp.float32),
        pltpu.VMEM_SHARED((NACC, HID), jnp.float32),
        pltpu.SemaphoreType.DMA,
        pltpu.SemaphoreType.DMA,
    ],
    compiler_params=pltpu.CompilerParams(use_tc_tiling_on_sc=False),
)
def _sc_degree(dstf, ones, zero, out, idxd, ones_v, acc, isem, ssem):
    cid = lax.axis_index("c")
    sid = lax.axis_index("s")
    wid = sid * 2 + cid
    pltpu.sync_copy(zero.at[pl.ds(sid * RPT, RPT)], acc.at[pl.ds(sid * RPT, RPT)])
    pltpu.sync_copy(ones, ones_v)
    plsc.subcore_barrier()
    base = wid * EW

    def idx_desc(r, g):
        return pltpu.make_async_copy(dstf.at[pl.ds(base + g * C, C)],
                                     idxd.at[r], isem)

    def scatter_desc(r):
        return pltpu.make_async_copy(ones_v, acc.at[idxd.at[r]], ssem)

    idx_desc(0, 0).start()
    idx_desc(0, 0).wait()
    idx_desc(1, 1).start()

    def body(g, carry):
        r = lax.rem(g, 3)
        r1 = lax.rem(g + 1, 3)
        r2 = lax.rem(g + 2, 3)

        @pl.when(g >= 1)
        def _():
            scatter_desc(r2).wait()

        scatter_desc(r).start(add=True)

        @pl.when(g <= NCH - 2)
        def _():
            idx_desc(r1, g + 1).wait()

        @pl.when(g <= NCH - 3)
        def _():
            idx_desc(r2, g + 2).start()
        return carry

    lax.fori_loop(0, NCH, body, 0)
    scatter_desc((NCH - 1) % 3).wait()
    plsc.subcore_barrier()
    pltpu.sync_copy(
        acc.at[pl.ds(sid * RPT, RPT)],
        out.at[pl.ds(cid * NACC + sid * RPT, RPT)],
    )


BM = 2000
GRID = N // BM
PHN = NACC * HID // 128

BL = PHN // 4
BLT = NACC // 4
GRID_L = 4


def _tc0_body(x_ref, w0_ref, h0_ref):
    h0_ref[...] = jnp.dot(x_ref[...], w0_ref[...],
                          preferred_element_type=jnp.float32)


def _tc0(x, w0):
    return pl.pallas_call(
        _tc0_body,
        grid=(GRID,),
        in_specs=[
            pl.BlockSpec((BM, IN_CH), lambda i: (i, 0)),
            pl.BlockSpec((IN_CH, HID), lambda i: (0, 0)),
        ],
        out_specs=pl.BlockSpec((BM, HID), lambda i: (i, 0)),
        out_shape=jax.ShapeDtypeStruct((N, HID), jnp.float32),
    )(x, w0)


def _pre_body(h0p_ref, dega_ref, degb_ref, dinv_ref, hp0_ref):
    dinv = lax.rsqrt(1.0 + dega_ref[...] + degb_ref[...])
    dinv_ref[...] = dinv
    hp0_ref[...] = h0p_ref[...] * dinv


def _tc_pre(h0p, dega, degb):
    return pl.pallas_call(
        _pre_body,
        grid=(GRID_L,),
        in_specs=[pl.BlockSpec((BL, 128), lambda i: (i, 0))] * 3,
        out_specs=[pl.BlockSpec((BL, 128), lambda i: (i, 0))] * 2,
        out_shape=[jax.ShapeDtypeStruct((PHN, 128), jnp.float32)] * 2,
    )(h0p, dega, degb)


def _ln_relu_p(conv, mmat, g, b):
    mu = jnp.dot(conv, mmat, preferred_element_type=jnp.float32)
    xc = conv - mu
    var = jnp.dot(xc * xc, mmat, preferred_element_type=jnp.float32)
    h = xc * lax.rsqrt(var + 1e-5) * g + b
    return jnp.maximum(h, 0.0)


def _layer_body(acca_ref, accb_ref, hp_ref, dinv_ref, mmat_ref, g_ref,
                bn_ref, bc_ref, wn_ref, h_ref, hpn_ref):
    dinv = dinv_ref[...]
    conv = dinv * (acca_ref[...] + accb_ref[...] + hp_ref[...]) + bc_ref[...]
    h = _ln_relu_p(conv, mmat_ref[...], g_ref[...], bn_ref[...])
    h_ref[...] = h
    hpn_ref[...] = (jnp.dot(h, wn_ref[...], preferred_element_type=jnp.float32)
                    * dinv)


def _tc_layer(acca, accb, hp, dinv, mmat, g, bn, bc, wn):
    return pl.pallas_call(
        _layer_body,
        grid=(GRID_L,),
        in_specs=[
            pl.BlockSpec((BL, 128), lambda i: (i, 0)),
            pl.BlockSpec((BL, 128), lambda i: (i, 0)),
            pl.BlockSpec((BL, 128), lambda i: (i, 0)),
            pl.BlockSpec((BL, 128), lambda i: (i, 0)),
            pl.BlockSpec((128, 128), lambda i: (0, 0)),
            pl.BlockSpec((1, 128), lambda i: (0, 0)),
            pl.BlockSpec((1, 128), lambda i: (0, 0)),
            pl.BlockSpec((1, 128), lambda i: (0, 0)),
            pl.BlockSpec((128, 128), lambda i: (0, 0)),
        ],
        out_specs=[pl.BlockSpec((BL, 128), lambda i: (i, 0))] * 2,
        out_shape=[jax.ShapeDtypeStruct((PHN, 128), jnp.float32)] * 2,
    )(acca, accb, hp, dinv, mmat, g, bn, bc, wn)


def _last_layer_body(acca_ref, accb_ref, hp_ref, dinv_ref, mmat_ref, g_ref,
                     bn_ref, bc_ref, h_ref):
    conv = (dinv_ref[...] * (acca_ref[...] + accb_ref[...] + hp_ref[...])
            + bc_ref[...])
    h_ref[...] = _ln_relu_p(conv, mmat_ref[...], g_ref[...], bn_ref[...])


def _tc_last_layer(acca, accb, hp, dinv, mmat, g, bn, bc):
    return pl.pallas_call(
        _last_layer_body,
        grid=(GRID_L,),
        in_specs=[
            pl.BlockSpec((BL, 128), lambda i: (i, 0)),
            pl.BlockSpec((BL, 128), lambda i: (i, 0)),
            pl.BlockSpec((BL, 128), lambda i: (i, 0)),
            pl.BlockSpec((BL, 128), lambda i: (i, 0)),
            pl.BlockSpec((128, 128), lambda i: (0, 0)),
            pl.BlockSpec((1, 128), lambda i: (0, 0)),
            pl.BlockSpec((1, 128), lambda i: (0, 0)),
            pl.BlockSpec((1, 128), lambda i: (0, 0)),
        ],
        out_specs=pl.BlockSpec((BL, 128), lambda i: (i, 0)),
        out_shape=jax.ShapeDtypeStruct((PHN, 128), jnp.float32),
    )(acca, accb, hp, dinv, mmat, g, bn, bc)


def _final_body(h1_ref, h2_ref, h3_ref, h4_ref, w2_ref, b2_ref, aw_ref,
                attb_ref, linw_ref, linb_ref, out_ref):
    s = [h1_ref[...], h2_ref[...], h3_ref[...], h4_ref[...]]
    w2 = w2_ref[...]
    b2 = b2_ref[...]
    H = LSTM_H
    hf = jnp.zeros((BM, H), jnp.float32)
    hb = jnp.zeros((BM, H), jnp.float32)
    cf = jnp.zeros((BM, H), jnp.float32)
    cb = jnp.zeros((BM, H), jnp.float32)
    fwd = [None] * 4
    bwd = [None] * 4
    for t in range(4):
        xt = jnp.concatenate([s[t], hf, s[3 - t], hb], axis=1)
        gates = jnp.dot(xt, w2, preferred_element_type=jnp.float32) + b2
        sg = jax.nn.sigmoid(gates[:, :192])
        tg = jnp.tanh(gates[:, 192:])
        cf = sg[:, 32:64] * cf + sg[:, 0:32] * tg[:, 0:32]
        cb = sg[:, 128:160] * cb + sg[:, 96:128] * tg[:, 32:64]
        tc2 = jnp.tanh(jnp.concatenate([cf, cb], axis=1))
        hf = sg[:, 64:96] * tc2[:, :H]
        hb = sg[:, 160:192] * tc2[:, H:]
        fwd[t] = hf
        bwd[3 - t] = hb
    lcat = jnp.concatenate(
        [jnp.concatenate([fwd[t], bwd[t]], axis=1) for t in range(4)], axis=1)
    logits = (jnp.dot(lcat, aw_ref[...], preferred_element_type=jnp.float32)
              + attb_ref[0, 0])
    m = jnp.max(logits, axis=1, keepdims=True)
    e = jnp.exp(logits - m)
    z = jnp.sum(e, axis=1, keepdims=True)
    out16 = sum(e[:, t:t + 1] * s[t] for t in range(4)) / z
    out_ref[...] = (jnp.dot(out16, linw_ref[...], preferred_element_type=jnp.float32)
                    + linb_ref[...])


def _tc_final(h1, h2, h3, h4, w2, b2, aw, attb, linw, linb):
    blk = lambda r, c: pl.BlockSpec((r, c), lambda i: (0, 0))
    return pl.pallas_call(
        _final_body,
        grid=(GRID,),
        in_specs=[
            pl.BlockSpec((BM, HID), lambda i: (i, 0)),
            pl.BlockSpec((BM, HID), lambda i: (i, 0)),
            pl.BlockSpec((BM, HID), lambda i: (i, 0)),
            pl.BlockSpec((BM, HID), lambda i: (i, 0)),
            blk(96, 256), blk(1, 256), blk(256, 4), blk(1, 1),
            blk(HID, OUT_CH), blk(1, OUT_CH),
        ],
        out_specs=pl.BlockSpec((BM, OUT_CH), lambda i: (i, 0)),
        out_shape=jax.ShapeDtypeStruct((N, OUT_CH), jnp.float32),
    )(h1, h2, h3, h4, w2, b2, aw, attb, linw, linb)


def kernel(x, edges, batch, w0, bc0, g0, bn0, w1, bc1, g1, bn1, w2, bc2, g2,
           bn2, w3, bc3, g3, bn3, lstm_wih, lstm_whh, lstm_bih, lstm_bhh,
           lstm_wih_r, lstm_whh_r, lstm_bih_r, lstm_bhh_r, att_w, att_b,
           lin_w, lin_b):
    src = edges[:, 0]
    dst = edges[:, 1]
    pad_dst = N + (jnp.arange(EPAD, dtype=jnp.int32) % DISCARD)
    srcf = jnp.concatenate([src, jnp.zeros((EPAD,), jnp.int32)])
    dstf = jnp.concatenate([dst, pad_dst])
    zero = jnp.zeros((NACC, HID), jnp.float32)
    ones = jnp.ones((C, HID), jnp.float32)

    def halves(o):
        op = o.reshape(2 * PHN, 128)
        return op[:PHN], op[PHN:]

    def unpack(a):
        return a.reshape(NACC, HID)

    eye8 = jnp.eye(8, dtype=jnp.float32)
    kron8 = lambda m: jnp.kron(eye8, m)
    mmat = kron8(jnp.full((HID, HID), 1.0 / HID, jnp.float32))
    tile8 = lambda v: jnp.tile(v, 8).reshape(1, 128)

    dega, degb = halves(_sc_degree(dstf, ones, zero))

    h0 = _tc0(x, w0)
    h0p = jnp.pad(h0.reshape(N * HID // 128, 128),
                  ((0, PHN - N * HID // 128), (0, 0)))
    dinv, hp = _tc_pre(h0p, dega, degb)

    hs = []
    for (g, bn, bc, wn) in ((g0, bn0, bc0, w1), (g1, bn1, bc1, w2),
                            (g2, bn2, bc2, w3)):
        acca, accb = halves(_sc_gather_scatter(srcf, dstf, unpack(hp), zero))
        h, hp = _tc_layer(acca, accb, hp, dinv, mmat, tile8(g), tile8(bn),
                          tile8(bc), kron8(wn))
        hs.append(unpack(h))
    acca, accb = halves(_sc_gather_scatter(srcf, dstf, unpack(hp), zero))
    hs.append(unpack(_tc_last_layer(acca, accb, hp, dinv, mmat, tile8(g3),
                                    tile8(bn3), tile8(bc3))))

    def lstm_pack(wih, whh, bih, bhh):
        wd = jnp.concatenate([wih.T, whh.T], axis=0)
        sig = jnp.concatenate([wd[:, 0:64], wd[:, 96:128]], axis=1)
        b = bih + bhh
        bsig = jnp.concatenate([b[0:64], b[96:128]])
        return sig, wd[:, 64:96], bsig, b[64:96]

    sigf, gf, bsf, bgf = lstm_pack(lstm_wih, lstm_whh, lstm_bih, lstm_bhh)
    sigb, gb, bsb, bgb = lstm_pack(lstm_wih_r, lstm_whh_r, lstm_bih_r,
                                   lstm_bhh_r)
    z96 = jnp.zeros((48, 96), jnp.float32)
    z32 = jnp.zeros((48, 32), jnp.float32)
    w2 = jnp.concatenate([
        jnp.concatenate([sigf, z96, gf, z32], axis=1),
        jnp.concatenate([z96, sigb, z32, gb], axis=1),
    ], axis=0)
    b2 = jnp.concatenate([bsf, bsb, bgf, bgb]).reshape(1, 256)
    aw = jnp.kron(jnp.eye(4, dtype=jnp.float32), att_w[0][:, None])

    out = _tc_final(hs[0], hs[1], hs[2], hs[3], w2, b2, aw,
                    att_b.reshape(1, 1), lin_w, lin_b.reshape(1, -1))
    return out


# speedup vs baseline: 41.5916x; 1.0962x over previous
import functools

import jax
import jax.numpy as jnp
from jax import lax
from jax.experimental import pallas as pl
from jax.experimental.pallas import tpu as pltpu
from jax.experimental.pallas import tpu_sc as plsc

N = 100000
E = 3200000
IN_CH = 896
HID = 16
OUT_CH = 640
LSTM_H = 32

NW = 32
EP = 3211264
EW = EP // NW
C = 512
NCH = EW // C
EPAD = EP - E
DISCARD = 352
NACC = N + DISCARD
RPT = NACC // 16

_mesh = plsc.VectorSubcoreMesh(core_axis_name="c", subcore_axis_name="s",
                               num_cores=2, num_subcores=16)


@functools.partial(
    pl.kernel,
    out_type=jax.ShapeDtypeStruct((2 * NACC, HID), jnp.float32),
    mesh=_mesh,
    scratch_types=[
        pltpu.VMEM((3, C), jnp.int32),
        pltpu.VMEM((3, C), jnp.int32),
        pltpu.VMEM((2, C, HID), jnp.float32),
        pltpu.VMEM_SHARED((NACC, HID), jnp.float32),
        pltpu.SemaphoreType.DMA,
        pltpu.SemaphoreType.DMA,
        pltpu.SemaphoreType.DMA,
    ],
    compiler_params=pltpu.CompilerParams(use_tc_tiling_on_sc=False),
)
def _sc_gather_scatter(srcf, dstf, feat, zero, out, idxs, idxd, rows, acc,
                       isem, gsem, ssem):
    cid = lax.axis_index("c")
    sid = lax.axis_index("s")
    wid = sid * 2 + cid
    pltpu.sync_copy(zero.at[pl.ds(sid * RPT, RPT)], acc.at[pl.ds(sid * RPT, RPT)])
    plsc.subcore_barrier()
    base = wid * EW

    def idx_descs(r, g):
        off = base + g * C
        return (
            pltpu.make_async_copy(srcf.at[pl.ds(off, C)], idxs.at[r], isem),
            pltpu.make_async_copy(dstf.at[pl.ds(off, C)], idxd.at[r], isem),
        )

    def gather_desc(r, p):
        return pltpu.make_async_copy(feat.at[idxs.at[r]], rows.at[p], gsem)

    def scatter_desc(r, p):
        return pltpu.make_async_copy(rows.at[p], acc.at[idxd.at[r]], ssem)

    for d in idx_descs(0, 0):
        d.start()
    for d in idx_descs(0, 0):
        d.wait()
    gather_desc(0, 0).start()
    for d in idx_descs(1, 1):
        d.start()

    def body(g, carry):
        p = lax.rem(g, 2)
        q = 1 - p
        r = lax.rem(g, 3)
        r1 = lax.rem(g + 1, 3)
        r2 = lax.rem(g + 2, 3)

        gather_desc(r, p).wait()

        @pl.when(g >= 1)
        def _():
            scatter_desc(r2, q).wait()

        scatter_desc(r, p).start(add=True)

        @pl.when(g <= NCH - 2)
        def _():
            for d in idx_descs(r1, g + 1):
                d.wait()
            gather_desc(r1, q).start()

        @pl.when(g <= NCH - 3)
        def _():
            for d in idx_descs(r2, g + 2):
                d.start()
        return carry

    lax.fori_loop(0, NCH, body, 0)
    scatter_desc((NCH - 1) % 3, (NCH - 1) % 2).wait()
    plsc.subcore_barrier()
    pltpu.sync_copy(
        acc.at[pl.ds(sid * RPT, RPT)],
        out.at[pl.ds(cid * NACC + sid * RPT, RPT)],
    )


@functools.partial(
    pl.kernel,
    out_type=jax.ShapeDtypeStruct((2 * NACC, HID), jnp.float32),
    mesh=_mesh,
    scratch_types=[
        pltpu.VMEM((3, C), jnp.int32),
        pltpu.VMEM((C, HID), jnp.float32),
        pltpu.VMEM_SHARED((NACC, HID), jnp.float32),
        pltpu.SemaphoreType.DMA,
        pltpu.SemaphoreType.DMA,
    ],
    compiler_params=pltpu.CompilerParams(use_tc_tiling_on_sc=False),
)
def _sc_degree(dstf, ones, zero, out, idxd, ones_v, acc, isem, ssem):
    cid = lax.axis_index("c")
    sid = lax.axis_index("s")
    wid = sid * 2 + cid
    pltpu.sync_copy(zero.at[pl.ds(sid * RPT, RPT)], acc.at[pl.ds(sid * RPT, RPT)])
    pltpu.sync_copy(ones, ones_v)
    plsc.subcore_barrier()
    base = wid * EW

    def idx_desc(r, g):
        return pltpu.make_async_copy(dstf.at[pl.ds(base + g * C, C)],
                                     idxd.at[r], isem)

    def scatter_desc(r):
        return pltpu.make_async_copy(ones_v, acc.at[idxd.at[r]], ssem)

    idx_desc(0, 0).start()
    idx_desc(0, 0).wait()
    idx_desc(1, 1).start()

    def body(g, carry):
        r = lax.rem(g, 3)
        r1 = lax.rem(g + 1, 3)
        r2 = lax.rem(g + 2, 3)

        @pl.when(g >= 1)
        def _():
            scatter_desc(r2).wait()

        scatter_desc(r).start(add=True)

        @pl.when(g <= NCH - 2)
        def _():
            idx_desc(r1, g + 1).wait()

        @pl.when(g <= NCH - 3)
        def _():
            idx_desc(r2, g + 2).start()
        return carry

    lax.fori_loop(0, NCH, body, 0)
    scatter_desc((NCH - 1) % 3).wait()
    plsc.subcore_barrier()
    pltpu.sync_copy(
        acc.at[pl.ds(sid * RPT, RPT)],
        out.at[pl.ds(cid * NACC + sid * RPT, RPT)],
    )


BM = 2000
GRID = N // BM
PHN = NACC * HID // 128

BL = PHN // 4
BLT = NACC // 4
GRID_L = 4


def _tc0_body(x_ref, w0_ref, h0_ref):
    h0_ref[...] = jnp.dot(x_ref[...], w0_ref[...],
                          preferred_element_type=jnp.float32)


def _tc0(x, w0):
    return pl.pallas_call(
        _tc0_body,
        grid=(GRID,),
        in_specs=[
            pl.BlockSpec((BM, IN_CH), lambda i: (i, 0)),
            pl.BlockSpec((IN_CH, HID), lambda i: (0, 0)),
        ],
        out_specs=pl.BlockSpec((BM, HID), lambda i: (i, 0)),
        out_shape=jax.ShapeDtypeStruct((N, HID), jnp.float32),
    )(x, w0)


def _pre_body(h0p_ref, dega_ref, degb_ref, dinv_ref, hp0_ref):
    dinv = lax.rsqrt(1.0 + dega_ref[...] + degb_ref[...])
    dinv_ref[...] = dinv
    hp0_ref[...] = h0p_ref[...] * dinv


def _tc_pre(h0p, dega, degb):
    return pl.pallas_call(
        _pre_body,
        grid=(GRID_L,),
        in_specs=[pl.BlockSpec((BL, 128), lambda i: (i, 0))] * 3,
        out_specs=[pl.BlockSpec((BL, 128), lambda i: (i, 0))] * 2,
        out_shape=[jax.ShapeDtypeStruct((PHN, 128), jnp.float32)] * 2,
    )(h0p, dega, degb)


def _ln_relu_p(conv, mmat, g, b):
    mu = jnp.dot(conv, mmat, preferred_element_type=jnp.float32)
    xc = conv - mu
    var = jnp.dot(xc * xc, mmat, preferred_element_type=jnp.float32)
    h = xc * lax.rsqrt(var + 1e-5) * g + b
    return jnp.maximum(h, 0.0)


def _layer_body(acca_ref, accb_ref, hp_ref, dinv_ref, mmat_ref, g_ref,
                bn_ref, bc_ref, wn_ref, h_ref, hpn_ref):
    dinv = dinv_ref[...]
    conv = dinv * (acca_ref[...] + accb_ref[...] + hp_ref[...]) + bc_ref[...]
    h = _ln_relu_p(conv, mmat_ref[...], g_ref[...], bn_ref[...])
    h_ref[...] = h
    hpn_ref[...] = (jnp.dot(h, wn_ref[...], preferred_element_type=jnp.float32)
                    * dinv)


def _tc_layer(acca, accb, hp, dinv, mmat, g, bn, bc, wn):
    return pl.pallas_call(
        _layer_body,
        grid=(GRID_L,),
        in_specs=[
            pl.BlockSpec((BL, 128), lambda i: (i, 0)),
            pl.BlockSpec((BL, 128), lambda i: (i, 0)),
            pl.BlockSpec((BL, 128), lambda i: (i, 0)),
            pl.BlockSpec((BL, 128), lambda i: (i, 0)),
            pl.BlockSpec((128, 128), lambda i: (0, 0)),
            pl.BlockSpec((1, 128), lambda i: (0, 0)),
            pl.BlockSpec((1, 128), lambda i: (0, 0)),
            pl.BlockSpec((1, 128), lambda i: (0, 0)),
            pl.BlockSpec((128, 128), lambda i: (0, 0)),
        ],
        out_specs=[pl.BlockSpec((BL, 128), lambda i: (i, 0))] * 2,
        out_shape=[jax.ShapeDtypeStruct((PHN, 128), jnp.float32)] * 2,
    )(acca, accb, hp, dinv, mmat, g, bn, bc, wn)


def _last_layer_body(acca_ref, accb_ref, hp_ref, dinv_ref, mmat_ref, g_ref,
                     bn_ref, bc_ref, h_ref):
    conv = (dinv_ref[...] * (acca_ref[...] + accb_ref[...] + hp_ref[...])
            + bc_ref[...])
    h_ref[...] = _ln_relu_p(conv, mmat_ref[...], g_ref[...], bn_ref[...])


def _tc_last_layer(acca, accb, hp, dinv, mmat, g, bn, bc):
    return pl.pallas_call(
        _last_layer_body,
        grid=(GRID_L,),
        in_specs=[
            pl.BlockSpec((BL, 128), lambda i: (i, 0)),
            pl.BlockSpec((BL, 128), lambda i: (i, 0)),
            pl.BlockSpec((BL, 128), lambda i: (i, 0)),
            pl.BlockSpec((BL, 128), lambda i: (i, 0)),
            pl.BlockSpec((128, 128), lambda i: (0, 0)),
            pl.BlockSpec((1, 128), lambda i: (0, 0)),
            pl.BlockSpec((1, 128), lambda i: (0, 0)),
            pl.BlockSpec((1, 128), lambda i: (0, 0)),
        ],
        out_specs=pl.BlockSpec((BL, 128), lambda i: (i, 0)),
        out_shape=jax.ShapeDtypeStruct((PHN, 128), jnp.float32),
    )(acca, accb, hp, dinv, mmat, g, bn, bc)


def _final_body(h1_ref, h2_ref, h3_ref, h4_ref, wsg_ref, whg_ref, bg_ref,
                awf_ref, awb_ref, attb_ref, linw_ref, linb_ref, out_ref):
    s = [h1_ref[...], h2_ref[...], h3_ref[...], h4_ref[...]]
    dot = lambda a, b: jnp.dot(a, b, preferred_element_type=jnp.float32)
    h = jnp.zeros((BM, 64), jnp.float32)
    c = jnp.zeros((BM, 64), jnp.float32)
    houts = [None] * 4
    for t in range(4):
        scat = jnp.concatenate([s[t], s[3 - t]], axis=1)

        def gate(k):
            return (dot(scat, wsg_ref[pl.ds(32 * k, 32), :])
                    + dot(h, whg_ref[pl.ds(64 * k, 64), :])
                    + bg_ref[k:k + 1, :])

        gi = jax.nn.sigmoid(gate(0))
        gf = jax.nn.sigmoid(gate(1))
        gg = jnp.tanh(gate(2))
        go = jax.nn.sigmoid(gate(3))
        c = gf * c + gi * gg
        h = go * jnp.tanh(c)
        houts[t] = h
    attb = attb_ref[0, 0]
    logits = [
        dot(houts[t], awf_ref[...]) + dot(houts[3 - t], awb_ref[...]) + attb
        for t in range(4)
    ]
    m = jnp.maximum(jnp.maximum(logits[0], logits[1]),
                    jnp.maximum(logits[2], logits[3]))
    es = [jnp.exp(l - m) for l in logits]
    z = es[0] + es[1] + es[2] + es[3]
    out16 = sum(es[t] * s[t] for t in range(4)) / z
    out_ref[...] = dot(out16, linw_ref[...]) + linb_ref[...]


def _tc_final(h1, h2, h3, h4, wsg, whg, bg, awf, awb, attb, linw, linb):
    blk = lambda r, c: pl.BlockSpec((r, c), lambda i: (0, 0))
    return pl.pallas_call(
        _final_body,
        grid=(GRID,),
        in_specs=[
            pl.BlockSpec((BM, HID), lambda i: (i, 0)),
            pl.BlockSpec((BM, HID), lambda i: (i, 0)),
            pl.BlockSpec((BM, HID), lambda i: (i, 0)),
            pl.BlockSpec((BM, HID), lambda i: (i, 0)),
            blk(128, 64), blk(256, 64), blk(4, 64),
            blk(64, 1), blk(64, 1), blk(1, 1),
            blk(HID, OUT_CH), blk(1, OUT_CH),
        ],
        out_specs=pl.BlockSpec((BM, OUT_CH), lambda i: (i, 0)),
        out_shape=jax.ShapeDtypeStruct((N, OUT_CH), jnp.float32),
    )(h1, h2, h3, h4, wsg, whg, bg, awf, awb, attb, linw, linb)


def kernel(x, edges, batch, w0, bc0, g0, bn0, w1, bc1, g1, bn1, w2, bc2, g2,
           bn2, w3, bc3, g3, bn3, lstm_wih, lstm_whh, lstm_bih, lstm_bhh,
           lstm_wih_r, lstm_whh_r, lstm_bih_r, lstm_bhh_r, att_w, att_b,
           lin_w, lin_b):
    src = edges[:, 0]
    dst = edges[:, 1]
    pad_dst = N + (jnp.arange(EPAD, dtype=jnp.int32) % DISCARD)
    srcf = jnp.concatenate([src, jnp.zeros((EPAD,), jnp.int32)])
    dstf = jnp.concatenate([dst, pad_dst])
    zero = jnp.zeros((NACC, HID), jnp.float32)
    ones = jnp.ones((C, HID), jnp.float32)

    def halves(o):
        op = o.reshape(2 * PHN, 128)
        return op[:PHN], op[PHN:]

    def unpack(a):
        return a.reshape(NACC, HID)

    eye8 = jnp.eye(8, dtype=jnp.float32)
    kron8 = lambda m: jnp.kron(eye8, m)
    mmat = kron8(jnp.full((HID, HID), 1.0 / HID, jnp.float32))
    tile8 = lambda v: jnp.tile(v, 8).reshape(1, 128)

    dega, degb = halves(_sc_degree(dstf, ones, zero))

    h0 = _tc0(x, w0)
    h0p = jnp.pad(h0.reshape(N * HID // 128, 128),
                  ((0, PHN - N * HID // 128), (0, 0)))
    dinv, hp = _tc_pre(h0p, dega, degb)

    hs = []
    for (g, bn, bc, wn) in ((g0, bn0, bc0, w1), (g1, bn1, bc1, w2),
                            (g2, bn2, bc2, w3)):
        acca, accb = halves(_sc_gather_scatter(srcf, dstf, unpack(hp), zero))
        h, hp = _tc_layer(acca, accb, hp, dinv, mmat, tile8(g), tile8(bn),
                          tile8(bc), kron8(wn))
        hs.append(unpack(h))
    acca, accb = halves(_sc_gather_scatter(srcf, dstf, unpack(hp), zero))
    hs.append(unpack(_tc_last_layer(acca, accb, hp, dinv, mmat, tile8(g3),
                                    tile8(bn3), tile8(bc3))))

    z16 = jnp.zeros((16, 32), jnp.float32)
    z32 = jnp.zeros((32, 32), jnp.float32)
    bf_all = lstm_bih + lstm_bhh
    bb_all = lstm_bih_r + lstm_bhh_r
    ws_blocks, wh_blocks, b_rows = [], [], []
    for k in range(4):
        r = slice(32 * k, 32 * k + 32)
        ws_blocks.append(jnp.concatenate([
            jnp.concatenate([lstm_wih[r].T, z16], axis=1),
            jnp.concatenate([z16, lstm_wih_r[r].T], axis=1),
        ], axis=0))
        wh_blocks.append(jnp.concatenate([
            jnp.concatenate([lstm_whh[r].T, z32], axis=1),
            jnp.concatenate([z32, lstm_whh_r[r].T], axis=1),
        ], axis=0))
        b_rows.append(jnp.concatenate([bf_all[r], bb_all[r]]))
    wsg = jnp.concatenate(ws_blocks, axis=0)
    whg = jnp.concatenate(wh_blocks, axis=0)
    bg = jnp.stack(b_rows, axis=0)
    z32c = jnp.zeros((32,), jnp.float32)
    awf = jnp.concatenate([att_w[0, :32], z32c]).reshape(64, 1)
    awb = jnp.concatenate([z32c, att_w[0, 32:]]).reshape(64, 1)

    out = _tc_final(hs[0], hs[1], hs[2], hs[3], wsg, whg, bg, awf, awb,
                    att_b.reshape(1, 1), lin_w, lin_b.reshape(1, -1))
    return out


# speedup vs baseline: 44.9478x; 1.0807x over previous
import functools

import jax
import jax.numpy as jnp
from jax import lax
from jax.experimental import pallas as pl
from jax.experimental.pallas import tpu as pltpu
from jax.experimental.pallas import tpu_sc as plsc

N = 100000
E = 3200000
IN_CH = 896
HID = 16
OUT_CH = 640
LSTM_H = 32

NW = 32
EP = 3211264
EW = EP // NW
C = 784
NCH = EW // C
EPAD = EP - E
DISCARD = 352
NACC = N + DISCARD
RPT = NACC // 16

_mesh = plsc.VectorSubcoreMesh(core_axis_name="c", subcore_axis_name="s",
                               num_cores=2, num_subcores=16)


@functools.partial(
    pl.kernel,
    out_type=jax.ShapeDtypeStruct((2 * NACC, HID), jnp.float32),
    mesh=_mesh,
    scratch_types=[
        pltpu.VMEM((3, C), jnp.int32),
        pltpu.VMEM((3, C), jnp.int32),
        pltpu.VMEM((2, C, HID), jnp.float32),
        pltpu.VMEM_SHARED((NACC, HID), jnp.float32),
        pltpu.SemaphoreType.DMA,
        pltpu.SemaphoreType.DMA,
        pltpu.SemaphoreType.DMA,
    ],
    compiler_params=pltpu.CompilerParams(use_tc_tiling_on_sc=False),
)
def _sc_gather_scatter(srcf, dstf, feat, zero, out, idxs, idxd, rows, acc,
                       isem, gsem, ssem):
    cid = lax.axis_index("c")
    sid = lax.axis_index("s")
    wid = sid * 2 + cid
    pltpu.sync_copy(zero.at[pl.ds(sid * RPT, RPT)], acc.at[pl.ds(sid * RPT, RPT)])
    plsc.subcore_barrier()
    base = wid * EW

    def idx_descs(r, g):
        off = base + g * C
        return (
            pltpu.make_async_copy(srcf.at[pl.ds(off, C)], idxs.at[r], isem),
            pltpu.make_async_copy(dstf.at[pl.ds(off, C)], idxd.at[r], isem),
        )

    def gather_desc(r, p):
        return pltpu.make_async_copy(feat.at[idxs.at[r]], rows.at[p], gsem)

    def scatter_desc(r, p):
        return pltpu.make_async_copy(rows.at[p], acc.at[idxd.at[r]], ssem)

    for d in idx_descs(0, 0):
        d.start()
    for d in idx_descs(0, 0):
        d.wait()
    gather_desc(0, 0).start()
    for d in idx_descs(1, 1):
        d.start()

    def body(g, carry):
        p = lax.rem(g, 2)
        q = 1 - p
        r = lax.rem(g, 3)
        r1 = lax.rem(g + 1, 3)
        r2 = lax.rem(g + 2, 3)

        gather_desc(r, p).wait()

        @pl.when(g >= 1)
        def _():
            scatter_desc(r2, q).wait()

        scatter_desc(r, p).start(add=True)

        @pl.when(g <= NCH - 2)
        def _():
            for d in idx_descs(r1, g + 1):
                d.wait()
            gather_desc(r1, q).start()

        @pl.when(g <= NCH - 3)
        def _():
            for d in idx_descs(r2, g + 2):
                d.start()
        return carry

    lax.fori_loop(0, NCH, body, 0)
    scatter_desc((NCH - 1) % 3, (NCH - 1) % 2).wait()
    plsc.subcore_barrier()
    pltpu.sync_copy(
        acc.at[pl.ds(sid * RPT, RPT)],
        out.at[pl.ds(cid * NACC + sid * RPT, RPT)],
    )


@functools.partial(
    pl.kernel,
    out_type=jax.ShapeDtypeStruct((2 * NACC, HID), jnp.float32),
    mesh=_mesh,
    scratch_types=[
        pltpu.VMEM((3, C), jnp.int32),
        pltpu.VMEM((C, HID), jnp.float32),
        pltpu.VMEM_SHARED((NACC, HID), jnp.float32),
        pltpu.SemaphoreType.DMA,
        pltpu.SemaphoreType.DMA,
    ],
    compiler_params=pltpu.CompilerParams(use_tc_tiling_on_sc=False),
)
def _sc_degree(dstf, ones, zero, out, idxd, ones_v, acc, isem, ssem):
    cid = lax.axis_index("c")
    sid = lax.axis_index("s")
    wid = sid * 2 + cid
    pltpu.sync_copy(zero.at[pl.ds(sid * RPT, RPT)], acc.at[pl.ds(sid * RPT, RPT)])
    pltpu.sync_copy(ones, ones_v)
    plsc.subcore_barrier()
    base = wid * EW

    def idx_desc(r, g):
        return pltpu.make_async_copy(dstf.at[pl.ds(base + g * C, C)],
                                     idxd.at[r], isem)

    def scatter_desc(r):
        return pltpu.make_async_copy(ones_v, acc.at[idxd.at[r]], ssem)

    idx_desc(0, 0).start()
    idx_desc(0, 0).wait()
    idx_desc(1, 1).start()

    def body(g, carry):
        r = lax.rem(g, 3)
        r1 = lax.rem(g + 1, 3)
        r2 = lax.rem(g + 2, 3)

        @pl.when(g >= 1)
        def _():
            scatter_desc(r2).wait()

        scatter_desc(r).start(add=True)

        @pl.when(g <= NCH - 2)
        def _():
            idx_desc(r1, g + 1).wait()

        @pl.when(g <= NCH - 3)
        def _():
            idx_desc(r2, g + 2).start()
        return carry

    lax.fori_loop(0, NCH, body, 0)
    scatter_desc((NCH - 1) % 3).wait()
    plsc.subcore_barrier()
    pltpu.sync_copy(
        acc.at[pl.ds(sid * RPT, RPT)],
        out.at[pl.ds(cid * NACC + sid * RPT, RPT)],
    )


BM = 2000
GRID = N // BM
PHN = NACC * HID // 128

BL = PHN // 4
BLT = NACC // 4
GRID_L = 4


def _tc0_body(x_ref, w0_ref, h0_ref):
    h0_ref[...] = jnp.dot(x_ref[...], w0_ref[...],
                          preferred_element_type=jnp.float32)


def _tc0(x, w0):
    return pl.pallas_call(
        _tc0_body,
        grid=(GRID,),
        in_specs=[
            pl.BlockSpec((BM, IN_CH), lambda i: (i, 0)),
            pl.BlockSpec((IN_CH, HID), lambda i: (0, 0)),
        ],
        out_specs=pl.BlockSpec((BM, HID), lambda i: (i, 0)),
        out_shape=jax.ShapeDtypeStruct((N, HID), jnp.float32),
    )(x, w0)


def _pre_body(h0p_ref, dega_ref, degb_ref, dinv_ref, hp0_ref):
    dinv = lax.rsqrt(1.0 + dega_ref[...] + degb_ref[...])
    dinv_ref[...] = dinv
    hp0_ref[...] = h0p_ref[...] * dinv


def _tc_pre(h0p, dega, degb):
    return pl.pallas_call(
        _pre_body,
        grid=(GRID_L,),
        in_specs=[pl.BlockSpec((BL, 128), lambda i: (i, 0))] * 3,
        out_specs=[pl.BlockSpec((BL, 128), lambda i: (i, 0))] * 2,
        out_shape=[jax.ShapeDtypeStruct((PHN, 128), jnp.float32)] * 2,
    )(h0p, dega, degb)


def _ln_relu_p(conv, mmat, g, b):
    mu = jnp.dot(conv, mmat, preferred_element_type=jnp.float32)
    xc = conv - mu
    var = jnp.dot(xc * xc, mmat, preferred_element_type=jnp.float32)
    h = xc * lax.rsqrt(var + 1e-5) * g + b
    return jnp.maximum(h, 0.0)


def _layer_body(acca_ref, accb_ref, hp_ref, dinv_ref, mmat_ref, g_ref,
                bn_ref, bc_ref, wn_ref, h_ref, hpn_ref):
    dinv = dinv_ref[...]
    conv = dinv * (acca_ref[...] + accb_ref[...] + hp_ref[...]) + bc_ref[...]
    h = _ln_relu_p(conv, mmat_ref[...], g_ref[...], bn_ref[...])
    h_ref[...] = h
    hpn_ref[...] = (jnp.dot(h, wn_ref[...], preferred_element_type=jnp.float32)
                    * dinv)


def _tc_layer(acca, accb, hp, dinv, mmat, g, bn, bc, wn):
    return pl.pallas_call(
        _layer_body,
        grid=(GRID_L,),
        in_specs=[
            pl.BlockSpec((BL, 128), lambda i: (i, 0)),
            pl.BlockSpec((BL, 128), lambda i: (i, 0)),
            pl.BlockSpec((BL, 128), lambda i: (i, 0)),
            pl.BlockSpec((BL, 128), lambda i: (i, 0)),
            pl.BlockSpec((128, 128), lambda i: (0, 0)),
            pl.BlockSpec((1, 128), lambda i: (0, 0)),
            pl.BlockSpec((1, 128), lambda i: (0, 0)),
            pl.BlockSpec((1, 128), lambda i: (0, 0)),
            pl.BlockSpec((128, 128), lambda i: (0, 0)),
        ],
        out_specs=[pl.BlockSpec((BL, 128), lambda i: (i, 0))] * 2,
        out_shape=[jax.ShapeDtypeStruct((PHN, 128), jnp.float32)] * 2,
    )(acca, accb, hp, dinv, mmat, g, bn, bc, wn)


def _last_layer_body(acca_ref, accb_ref, hp_ref, dinv_ref, mmat_ref, g_ref,
                     bn_ref, bc_ref, h_ref):
    conv = (dinv_ref[...] * (acca_ref[...] + accb_ref[...] + hp_ref[...])
            + bc_ref[...])
    h_ref[...] = _ln_relu_p(conv, mmat_ref[...], g_ref[...], bn_ref[...])


def _tc_last_layer(acca, accb, hp, dinv, mmat, g, bn, bc):
    return pl.pallas_call(
        _last_layer_body,
        grid=(GRID_L,),
        in_specs=[
            pl.BlockSpec((BL, 128), lambda i: (i, 0)),
            pl.BlockSpec((BL, 128), lambda i: (i, 0)),
            pl.BlockSpec((BL, 128), lambda i: (i, 0)),
            pl.BlockSpec((BL, 128), lambda i: (i, 0)),
            pl.BlockSpec((128, 128), lambda i: (0, 0)),
            pl.BlockSpec((1, 128), lambda i: (0, 0)),
            pl.BlockSpec((1, 128), lambda i: (0, 0)),
            pl.BlockSpec((1, 128), lambda i: (0, 0)),
        ],
        out_specs=pl.BlockSpec((BL, 128), lambda i: (i, 0)),
        out_shape=jax.ShapeDtypeStruct((PHN, 128), jnp.float32),
    )(acca, accb, hp, dinv, mmat, g, bn, bc)


def _final_body(h1_ref, h2_ref, h3_ref, h4_ref, wsg_ref, whg_ref, bg_ref,
                awf_ref, awb_ref, attb_ref, linw_ref, linb_ref, out_ref):
    s = [h1_ref[...], h2_ref[...], h3_ref[...], h4_ref[...]]
    dot = lambda a, b: jnp.dot(a, b, preferred_element_type=jnp.float32)
    h = jnp.zeros((BM, 64), jnp.float32)
    c = jnp.zeros((BM, 64), jnp.float32)
    houts = [None] * 4
    for t in range(4):
        scat = jnp.concatenate([s[t], s[3 - t]], axis=1)

        def gate(k):
            return (dot(scat, wsg_ref[pl.ds(32 * k, 32), :])
                    + dot(h, whg_ref[pl.ds(64 * k, 64), :])
                    + bg_ref[k:k + 1, :])

        gi = jax.nn.sigmoid(gate(0))
        gf = jax.nn.sigmoid(gate(1))
        gg = jnp.tanh(gate(2))
        go = jax.nn.sigmoid(gate(3))
        c = gf * c + gi * gg
        h = go * jnp.tanh(c)
        houts[t] = h
    attb = attb_ref[0, 0]
    logits = [
        dot(houts[t], awf_ref[...]) + dot(houts[3 - t], awb_ref[...]) + attb
        for t in range(4)
    ]
    m = jnp.maximum(jnp.maximum(logits[0], logits[1]),
                    jnp.maximum(logits[2], logits[3]))
    es = [jnp.exp(l - m) for l in logits]
    z = es[0] + es[1] + es[2] + es[3]
    out16 = sum(es[t] * s[t] for t in range(4)) / z
    out_ref[...] = dot(out16, linw_ref[...]) + linb_ref[...]


def _tc_final(h1, h2, h3, h4, wsg, whg, bg, awf, awb, attb, linw, linb):
    blk = lambda r, c: pl.BlockSpec((r, c), lambda i: (0, 0))
    return pl.pallas_call(
        _final_body,
        grid=(GRID,),
        in_specs=[
            pl.BlockSpec((BM, HID), lambda i: (i, 0)),
            pl.BlockSpec((BM, HID), lambda i: (i, 0)),
            pl.BlockSpec((BM, HID), lambda i: (i, 0)),
            pl.BlockSpec((BM, HID), lambda i: (i, 0)),
            blk(128, 64), blk(256, 64), blk(4, 64),
            blk(64, 1), blk(64, 1), blk(1, 1),
            blk(HID, OUT_CH), blk(1, OUT_CH),
        ],
        out_specs=pl.BlockSpec((BM, OUT_CH), lambda i: (i, 0)),
        out_shape=jax.ShapeDtypeStruct((N, OUT_CH), jnp.float32),
    )(h1, h2, h3, h4, wsg, whg, bg, awf, awb, attb, linw, linb)


def kernel(x, edges, batch, w0, bc0, g0, bn0, w1, bc1, g1, bn1, w2, bc2, g2,
           bn2, w3, bc3, g3, bn3, lstm_wih, lstm_whh, lstm_bih, lstm_bhh,
           lstm_wih_r, lstm_whh_r, lstm_bih_r, lstm_bhh_r, att_w, att_b,
           lin_w, lin_b):
    src = edges[:, 0]
    dst = edges[:, 1]
    pad_dst = N + (jnp.arange(EPAD, dtype=jnp.int32) % DISCARD)
    srcf = jnp.concatenate([src, jnp.zeros((EPAD,), jnp.int32)])
    dstf = jnp.concatenate([dst, pad_dst])
    zero = jnp.zeros((NACC, HID), jnp.float32)
    ones = jnp.ones((C, HID), jnp.float32)

    def halves(o):
        op = o.reshape(2 * PHN, 128)
        return op[:PHN], op[PHN:]

    def unpack(a):
        return a.reshape(NACC, HID)

    eye8 = jnp.eye(8, dtype=jnp.float32)
    kron8 = lambda m: jnp.kron(eye8, m)
    mmat = kron8(jnp.full((HID, HID), 1.0 / HID, jnp.float32))
    tile8 = lambda v: jnp.tile(v, 8).reshape(1, 128)

    dega, degb = halves(_sc_degree(dstf, ones, zero))

    h0 = _tc0(x, w0)
    h0p = jnp.pad(h0.reshape(N * HID // 128, 128),
                  ((0, PHN - N * HID // 128), (0, 0)))
    dinv, hp = _tc_pre(h0p, dega, degb)

    hs = []
    for (g, bn, bc, wn) in ((g0, bn0, bc0, w1), (g1, bn1, bc1, w2),
                            (g2, bn2, bc2, w3)):
        acca, accb = halves(_sc_gather_scatter(srcf, dstf, unpack(hp), zero))
        h, hp = _tc_layer(acca, accb, hp, dinv, mmat, tile8(g), tile8(bn),
                          tile8(bc), kron8(wn))
        hs.append(unpack(h))
    acca, accb = halves(_sc_gather_scatter(srcf, dstf, unpack(hp), zero))
    hs.append(unpack(_tc_last_layer(acca, accb, hp, dinv, mmat, tile8(g3),
                                    tile8(bn3), tile8(bc3))))

    z16 = jnp.zeros((16, 32), jnp.float32)
    z32 = jnp.zeros((32, 32), jnp.float32)
    bf_all = lstm_bih + lstm_bhh
    bb_all = lstm_bih_r + lstm_bhh_r
    ws_blocks, wh_blocks, b_rows = [], [], []
    for k in range(4):
        r = slice(32 * k, 32 * k + 32)
        ws_blocks.append(jnp.concatenate([
            jnp.concatenate([lstm_wih[r].T, z16], axis=1),
            jnp.concatenate([z16, lstm_wih_r[r].T], axis=1),
        ], axis=0))
        wh_blocks.append(jnp.concatenate([
            jnp.concatenate([lstm_whh[r].T, z32], axis=1),
            jnp.concatenate([z32, lstm_whh_r[r].T], axis=1),
        ], axis=0))
        b_rows.append(jnp.concatenate([bf_all[r], bb_all[r]]))
    wsg = jnp.concatenate(ws_blocks, axis=0)
    whg = jnp.concatenate(wh_blocks, axis=0)
    bg = jnp.stack(b_rows, axis=0)
    z32c = jnp.zeros((32,), jnp.float32)
    awf = jnp.concatenate([att_w[0, :32], z32c]).reshape(64, 1)
    awb = jnp.concatenate([z32c, att_w[0, 32:]]).reshape(64, 1)

    out = _tc_final(hs[0], hs[1], hs[2], hs[3], wsg, whg, bg, awf, awb,
                    att_b.reshape(1, 1), lin_w, lin_b.reshape(1, -1))
    return out
